# Initial kernel scaffold; baseline (speedup 1.0000x reference)
import functools
import math

import jax
import jax.numpy as jnp
from jax import lax
from jax.experimental import pallas as pl
from jax.experimental.pallas import tpu as pltpu
from jax.experimental.pallas import tpu_sc as plsc

_N = 10000
_NPAD = 10240
_E = 320000
_NC = 2
_NS = 16
_NW = _NC * _NS
_CHUNK = 128
_EPAD = 327680
_ECHUNKS = _EPAD // (_NW * _CHUNK)
_ROWS_PT = _NPAD // _NS
_P = 65536
_PCHUNKS = _P // (_NW * _CHUNK)
_BN_SCALE = 1.0 / math.sqrt(1.0 + 1e-5)

_mesh = plsc.VectorSubcoreMesh(core_axis_name="c", subcore_axis_name="s")


def _deg_kernel():
    @functools.partial(
        pl.kernel,
        out_type=jax.ShapeDtypeStruct((_NC, _NPAD, 16), jnp.float32),
        mesh=_mesh,
        scratch_types=[
            pltpu.VMEM((_CHUNK,), jnp.int32),
            pltpu.VMEM((_CHUNK, 16), jnp.float32),
            pltpu.VMEM_SHARED((_NPAD, 16), jnp.float32),
        ],
    )
    def k(dst_hbm, e0_hbm, z_hbm, out_hbm, dst_v, e0_v, acc_sh):
        c = lax.axis_index("c")
        s = lax.axis_index("s")
        wid = c * _NS + s
        row0 = s * _ROWS_PT
        pltpu.sync_copy(z_hbm, acc_sh.at[pl.ds(row0, _ROWS_PT)])
        pltpu.sync_copy(e0_hbm, e0_v)
        plsc.subcore_barrier()

        def body(i, carry):
            off = pl.multiple_of((wid * _ECHUNKS + i) * _CHUNK, _CHUNK)
            pltpu.sync_copy(dst_hbm.at[pl.ds(off, _CHUNK)], dst_v)
            pltpu.sync_copy(e0_v, acc_sh.at[dst_v], add=True)
            return carry

        lax.fori_loop(0, _ECHUNKS, body, 0)
        plsc.subcore_barrier()
        pltpu.sync_copy(acc_sh.at[pl.ds(row0, _ROWS_PT)],
                        out_hbm.at[c].at[pl.ds(row0, _ROWS_PT)])

    return k


def _scatter_kernel(d):
    @functools.partial(
        pl.kernel,
        out_type=jax.ShapeDtypeStruct((_NC, _NPAD, d), jnp.float32),
        mesh=_mesh,
        scratch_types=[
            pltpu.VMEM((_CHUNK,), jnp.int32),
            pltpu.VMEM((_CHUNK,), jnp.int32),
            pltpu.VMEM((_CHUNK, d), jnp.float32),
            pltpu.VMEM_SHARED((_NPAD, d), jnp.float32),
            pltpu.SemaphoreType.DMA,
        ],
    )
    def k(h_hbm, src_hbm, dst_hbm, z_hbm, out_hbm, src_v, dst_v, rows_v,
          acc_sh, sem):
        c = lax.axis_index("c")
        s = lax.axis_index("s")
        wid = c * _NS + s
        row0 = s * _ROWS_PT
        pltpu.sync_copy(z_hbm, acc_sh.at[pl.ds(row0, _ROWS_PT)])
        plsc.subcore_barrier()

        def body(i, carry):
            off = pl.multiple_of((wid * _ECHUNKS + i) * _CHUNK, _CHUNK)
            pltpu.sync_copy(src_hbm.at[pl.ds(off, _CHUNK)], src_v)
            pltpu.sync_copy(dst_hbm.at[pl.ds(off, _CHUNK)], dst_v)
            pltpu.async_copy(h_hbm.at[src_v], rows_v, sem).wait()
            pltpu.sync_copy(rows_v, acc_sh.at[dst_v], add=True)
            return carry

        lax.fori_loop(0, _ECHUNKS, body, 0)
        plsc.subcore_barrier()
        pltpu.sync_copy(acc_sh.at[pl.ds(row0, _ROWS_PT)],
                        out_hbm.at[c].at[pl.ds(row0, _ROWS_PT)])

    return k


def _pairs_kernel():
    sds = jax.ShapeDtypeStruct((_P, 64), jnp.float32)

    @functools.partial(
        pl.kernel,
        out_type=(sds, sds),
        mesh=_mesh,
        scratch_types=[
            pltpu.VMEM((_CHUNK,), jnp.int32),
            pltpu.VMEM((_CHUNK,), jnp.int32),
            pltpu.VMEM((_CHUNK, 64), jnp.float32),
            pltpu.VMEM((_CHUNK, 64), jnp.float32),
            pltpu.SemaphoreType.DMA,
            pltpu.SemaphoreType.DMA,
        ],
    )
    def k(p_hbm, q_hbm, src_hbm, dst_hbm, outr_hbm, outs_hbm,
          src_v, dst_v, bufp, bufq, semp, semq):
        c = lax.axis_index("c")
        s = lax.axis_index("s")
        wid = c * _NS + s

        def body(i, carry):
            off = pl.multiple_of((wid * _PCHUNKS + i) * _CHUNK, _CHUNK)
            pltpu.sync_copy(src_hbm.at[pl.ds(off, _CHUNK)], src_v)
            pltpu.sync_copy(dst_hbm.at[pl.ds(off, _CHUNK)], dst_v)
            cp = pltpu.async_copy(p_hbm.at[src_v], bufp, semp)
            cq = pltpu.async_copy(q_hbm.at[dst_v], bufq, semq)
            cp.wait()
            cq.wait()
            pltpu.sync_copy(bufp, outr_hbm.at[pl.ds(off, _CHUNK)])
            pltpu.sync_copy(bufq, outs_hbm.at[pl.ds(off, _CHUNK)])
            return carry

        lax.fori_loop(0, _PCHUNKS, body, 0)

    return k


def _tc_prep(degp, x, w1):
    def body(deg_ref, x_ref, w_ref, dinv_ref, h1s_ref):
        deg16 = deg_ref[0] + deg_ref[1] + 1.0
        dinv16 = lax.rsqrt(deg16)
        dinv_ref[...] = dinv16[:, 0:1]
        h = jnp.dot(x_ref[...], w_ref[...],
                    preferred_element_type=jnp.float32)
        h1s_ref[...] = dinv16[:_N, 0:1] * h

    return pl.pallas_call(
        body,
        out_shape=(jax.ShapeDtypeStruct((_NPAD, 1), jnp.float32),
                   jax.ShapeDtypeStruct((_N, 128), jnp.float32)),
    )(degp, x, w1)


def _tc_mid(acc1, h1s, dinv, b1, g1, be1, w2):
    def body(acc_ref, h_ref, dinv_ref, b_ref, g_ref, be_ref, w_ref, out_ref):
        dv = dinv_ref[:_N]
        o1 = dv * (acc_ref[0, :_N] + acc_ref[1, :_N] + h_ref[...]) + b_ref[...]
        z1 = jnp.maximum(g_ref[...] * (o1 * _BN_SCALE) + be_ref[...], 0.0)
        out_ref[...] = dv * jnp.dot(z1, w_ref[...],
                                    preferred_element_type=jnp.float32)

    return pl.pallas_call(
        body,
        out_shape=jax.ShapeDtypeStruct((_N, 64), jnp.float32),
    )(acc1, h1s, dinv, b1, g1, be1, w2)


def _tc_head(acc2, h2s, dinv, b2, g2, be2, wh1):
    def body(acc_ref, h_ref, dinv_ref, b_ref, g_ref, be_ref, w_ref,
             p_ref, q_ref):
        dv = dinv_ref[:_N]
        o2 = dv * (acc_ref[0, :_N] + acc_ref[1, :_N] + h_ref[...]) + b_ref[...]
        z2 = jnp.maximum(g_ref[...] * (o2 * _BN_SCALE) + be_ref[...], 0.0)
        w = w_ref[...]
        p_ref[...] = jnp.dot(z2, w[:64], preferred_element_type=jnp.float32)
        q_ref[...] = jnp.dot(z2, w[64:], preferred_element_type=jnp.float32)

    return pl.pallas_call(
        body,
        out_shape=(jax.ShapeDtypeStruct((_N, 64), jnp.float32),
                   jax.ShapeDtypeStruct((_N, 64), jnp.float32)),
    )(acc2, h2s, dinv, b2, g2, be2, wh1)


def _tc_final(r, s, bh1, wh2, bh2):
    blk = 8192

    def body(r_ref, s_ref, b_ref, w_ref, b2_ref, out_ref):
        h = jnp.maximum(r_ref[...] + s_ref[...] + b_ref[...], 0.0)
        acc = jnp.sum(h * w_ref[...], axis=1, keepdims=True) + b2_ref[0, 0]
        out_ref[...] = 1.0 / (1.0 + jnp.exp(-acc))

    grid = _P // blk
    return pl.pallas_call(
        body,
        grid=(grid,),
        in_specs=[
            pl.BlockSpec((blk, 64), lambda i: (i, 0)),
            pl.BlockSpec((blk, 64), lambda i: (i, 0)),
            pl.BlockSpec((1, 64), lambda i: (0, 0)),
            pl.BlockSpec((1, 64), lambda i: (0, 0)),
            pl.BlockSpec((1, 1), lambda i: (0, 0)),
        ],
        out_specs=pl.BlockSpec((blk, 1), lambda i: (i, 0)),
        out_shape=jax.ShapeDtypeStruct((_P, 1), jnp.float32),
    )(r, s, bh1, wh2, bh2)


def kernel(x, ei, src, dst, W1, b1, g1, be1, W2, b2, g2, be2,
           Wh1, bh1, Wh2, bh2):
    npd = _EPAD - _E
    src_pad = jnp.concatenate([ei[0], jnp.zeros((npd,), jnp.int32)])
    dst_pad = jnp.concatenate([ei[1], jnp.full((npd,), _N, jnp.int32)])

    e0 = jnp.zeros((_CHUNK, 16), jnp.float32).at[:, 0].set(1.0)
    z16 = jnp.zeros((_ROWS_PT, 16), jnp.float32)
    z128 = jnp.zeros((_ROWS_PT, 128), jnp.float32)
    z64 = jnp.zeros((_ROWS_PT, 64), jnp.float32)

    degp = _deg_kernel()(dst_pad, e0, z16)
    dinv, h1s = _tc_prep(degp, x, W1)
    acc1 = _scatter_kernel(128)(h1s, src_pad, dst_pad, z128)
    h2s = _tc_mid(acc1, h1s, dinv, b1.reshape(1, -1), g1.reshape(1, -1),
                  be1.reshape(1, -1), W2)
    acc2 = _scatter_kernel(64)(h2s, src_pad, dst_pad, z64)
    p, q = _tc_head(acc2, h2s, dinv, b2.reshape(1, -1), g2.reshape(1, -1),
                    be2.reshape(1, -1), Wh1)
    r, s = _pairs_kernel()(p, q, src, dst)
    out = _tc_final(r, s, bh1.reshape(1, -1), Wh2.reshape(1, -1),
                    bh2.reshape(1, 1))
    return out[:, 0]


# speedup vs baseline: 6.8387x; 6.8387x over previous
import functools
import math

import jax
import jax.numpy as jnp
from jax import lax
from jax.experimental import pallas as pl
from jax.experimental.pallas import tpu as pltpu
from jax.experimental.pallas import tpu_sc as plsc

_N = 10000
_NPAD = 10240
_E = 320000
_NC = 2
_NS = 16
_NW = _NC * _NS
_CHUNK = 128
_EPAD = 327680
_ECHUNKS = _EPAD // (_NW * _CHUNK)
_ROWS_PT = _NPAD // _NS
_P = 65536
_PCHUNKS = _P // (_NW * _CHUNK)
_DROWS = _NPAD // 128
_BN_SCALE = 1.0 / math.sqrt(1.0 + 1e-5)

_mesh = plsc.VectorSubcoreMesh(core_axis_name="c", subcore_axis_name="s")


def _deg_kernel():
    @functools.partial(
        pl.kernel,
        out_type=jax.ShapeDtypeStruct((_NC, _NPAD, 128), jnp.float32),
        mesh=_mesh,
        scratch_types=[
            pltpu.VMEM((_CHUNK,), jnp.int32),
            pltpu.VMEM((_CHUNK, 128), jnp.float32),
            pltpu.VMEM_SHARED((_NPAD, 128), jnp.float32),
        ],
    )
    def k(dst_hbm, ones_hbm, z_hbm, out_hbm, dst_v, ones_v, acc_sh):
        c = lax.axis_index("c")
        s = lax.axis_index("s")
        wid = c * _NS + s
        row0 = s * _ROWS_PT
        pltpu.sync_copy(z_hbm, acc_sh.at[pl.ds(row0, _ROWS_PT)])
        pltpu.sync_copy(ones_hbm, ones_v)
        plsc.subcore_barrier()

        def body(i, carry):
            off = pl.multiple_of((wid * _ECHUNKS + i) * _CHUNK, _CHUNK)
            pltpu.sync_copy(dst_hbm.at[pl.ds(off, _CHUNK)], dst_v)
            pltpu.sync_copy(ones_v, acc_sh.at[dst_v], add=True)
            return carry

        lax.fori_loop(0, _ECHUNKS, body, 0)
        plsc.subcore_barrier()
        pltpu.sync_copy(acc_sh.at[pl.ds(row0, _ROWS_PT)],
                        out_hbm.at[c].at[pl.ds(row0, _ROWS_PT)])

    return k


def _scatter_kernel():
    @functools.partial(
        pl.kernel,
        out_type=jax.ShapeDtypeStruct((_NC, _NPAD, 128), jnp.float32),
        mesh=_mesh,
        scratch_types=[
            pltpu.VMEM((_CHUNK,), jnp.int32),
            pltpu.VMEM((_CHUNK,), jnp.int32),
            pltpu.VMEM((_CHUNK, 128), jnp.float32),
            pltpu.VMEM_SHARED((_NPAD, 128), jnp.float32),
            pltpu.SemaphoreType.DMA,
        ],
    )
    def k(h_hbm, src_hbm, dst_hbm, z_hbm, out_hbm, src_v, dst_v, rows_v,
          acc_sh, sem):
        c = lax.axis_index("c")
        s = lax.axis_index("s")
        wid = c * _NS + s
        row0 = s * _ROWS_PT
        pltpu.sync_copy(z_hbm, acc_sh.at[pl.ds(row0, _ROWS_PT)])
        plsc.subcore_barrier()

        def body(i, carry):
            off = pl.multiple_of((wid * _ECHUNKS + i) * _CHUNK, _CHUNK)
            pltpu.sync_copy(src_hbm.at[pl.ds(off, _CHUNK)], src_v)
            pltpu.sync_copy(dst_hbm.at[pl.ds(off, _CHUNK)], dst_v)
            pltpu.async_copy(h_hbm.at[src_v], rows_v, sem).wait()
            pltpu.sync_copy(rows_v, acc_sh.at[dst_v], add=True)
            return carry

        lax.fori_loop(0, _ECHUNKS, body, 0)
        plsc.subcore_barrier()
        pltpu.sync_copy(acc_sh.at[pl.ds(row0, _ROWS_PT)],
                        out_hbm.at[c].at[pl.ds(row0, _ROWS_PT)])

    return k


def _pairs_kernel():
    sds = jax.ShapeDtypeStruct((_P, 128), jnp.float32)

    @functools.partial(
        pl.kernel,
        out_type=(sds, sds),
        mesh=_mesh,
        scratch_types=[
            pltpu.VMEM((_CHUNK,), jnp.int32),
            pltpu.VMEM((_CHUNK,), jnp.int32),
            pltpu.VMEM((_CHUNK, 128), jnp.float32),
            pltpu.VMEM((_CHUNK, 128), jnp.float32),
            pltpu.SemaphoreType.DMA,
            pltpu.SemaphoreType.DMA,
        ],
    )
    def k(pq_hbm, qp_hbm, src_hbm, dst_hbm, outr_hbm, outs_hbm,
          src_v, dst_v, bufp, bufq, semp, semq):
        c = lax.axis_index("c")
        s = lax.axis_index("s")
        wid = c * _NS + s

        def body(i, carry):
            off = pl.multiple_of((wid * _PCHUNKS + i) * _CHUNK, _CHUNK)
            pltpu.sync_copy(src_hbm.at[pl.ds(off, _CHUNK)], src_v)
            pltpu.sync_copy(dst_hbm.at[pl.ds(off, _CHUNK)], dst_v)
            cp = pltpu.async_copy(pq_hbm.at[src_v], bufp, semp)
            cq = pltpu.async_copy(qp_hbm.at[dst_v], bufq, semq)
            cp.wait()
            cq.wait()
            pltpu.sync_copy(bufp, outr_hbm.at[pl.ds(off, _CHUNK)])
            pltpu.sync_copy(bufq, outs_hbm.at[pl.ds(off, _CHUNK)])
            return carry

        lax.fori_loop(0, _PCHUNKS, body, 0)

    return k


def _tc_prep(degp, x, w1):
    def body(deg_ref, x_ref, w_ref, dinv_ref, h1s_ref):
        dinv = lax.rsqrt(deg_ref[0] + deg_ref[1] + 1.0)
        dinv_ref[...] = dinv
        h = jnp.dot(x_ref[...], w_ref[...],
                    preferred_element_type=jnp.float32)
        h1s_ref[...] = dinv[:_N] * h

    return pl.pallas_call(
        body,
        out_shape=(jax.ShapeDtypeStruct((_NPAD, 128), jnp.float32),
                   jax.ShapeDtypeStruct((_N, 128), jnp.float32)),
    )(degp, x, w1)


def _tc_mid(acc1, h1s, dinv, b1, g1, be1, w2p):
    def body(acc_ref, h_ref, dinv_ref, b_ref, g_ref, be_ref, w_ref, out_ref):
        dv = dinv_ref[:_N]
        o1 = dv * (acc_ref[0, :_N] + acc_ref[1, :_N] + h_ref[...]) + b_ref[...]
        z1 = jnp.maximum(g_ref[...] * (o1 * _BN_SCALE) + be_ref[...], 0.0)
        out_ref[...] = dv * jnp.dot(z1, w_ref[...],
                                    preferred_element_type=jnp.float32)

    return pl.pallas_call(
        body,
        out_shape=jax.ShapeDtypeStruct((_N, 128), jnp.float32),
    )(acc1, h1s, dinv, b1, g1, be1, w2p)


def _tc_head(acc2, h2s, dinv, b2, g2, be2, whp, whq):
    def body(acc_ref, h_ref, dinv_ref, b_ref, g_ref, be_ref, wp_ref, wq_ref,
             pq_ref, qp_ref):
        dv = dinv_ref[:_N]
        o2 = dv * (acc_ref[0, :_N] + acc_ref[1, :_N] + h_ref[...]) + b_ref[...]
        z2 = jnp.maximum(g_ref[...] * (o2 * _BN_SCALE) + be_ref[...], 0.0)
        pq_ref[...] = jnp.dot(z2, wp_ref[...],
                              preferred_element_type=jnp.float32)
        qp_ref[...] = jnp.dot(z2, wq_ref[...],
                              preferred_element_type=jnp.float32)

    return pl.pallas_call(
        body,
        out_shape=(jax.ShapeDtypeStruct((_N, 128), jnp.float32),
                   jax.ShapeDtypeStruct((_N, 128), jnp.float32)),
    )(acc2, h2s, dinv, b2, g2, be2, whp, whq)


def _tc_final(r, s, bh1p, wh2p, bh2):
    blk = 8192

    def body(r_ref, s_ref, b_ref, w_ref, b2_ref, out_ref):
        h = jnp.maximum(r_ref[...] + s_ref[...] + b_ref[...], 0.0)
        acc = jnp.sum(h * w_ref[...], axis=1, keepdims=True) + b2_ref[0, 0]
        out_ref[...] = 1.0 / (1.0 + jnp.exp(-acc))

    grid = _P // blk
    return pl.pallas_call(
        body,
        grid=(grid,),
        in_specs=[
            pl.BlockSpec((blk, 128), lambda i: (i, 0)),
            pl.BlockSpec((blk, 128), lambda i: (i, 0)),
            pl.BlockSpec((1, 128), lambda i: (0, 0)),
            pl.BlockSpec((1, 128), lambda i: (0, 0)),
            pl.BlockSpec((1, 1), lambda i: (0, 0)),
        ],
        out_specs=pl.BlockSpec((blk, 1), lambda i: (i, 0)),
        out_shape=jax.ShapeDtypeStruct((_P, 1), jnp.float32),
    )(r, s, bh1p, wh2p, bh2)


def kernel(x, ei, src, dst, W1, b1, g1, be1, W2, b2, g2, be2,
           Wh1, bh1, Wh2, bh2):
    npd = _EPAD - _E
    src_pad = jnp.concatenate([ei[0], jnp.zeros((npd,), jnp.int32)])
    dst_pad = jnp.concatenate([ei[1], jnp.full((npd,), _N, jnp.int32)])

    ones128 = jnp.ones((_CHUNK, 128), jnp.float32)
    z128 = jnp.zeros((_ROWS_PT, 128), jnp.float32)

    w2p = jnp.concatenate([W2, jnp.zeros((128, 64), jnp.float32)], axis=1)
    b1r, g1r, be1r = (a.reshape(1, -1) for a in (b1, g1, be1))
    pad64 = jnp.zeros((1, 64), jnp.float32)
    b2p = jnp.concatenate([b2.reshape(1, -1), pad64], axis=1)
    g2p = jnp.concatenate([g2.reshape(1, -1), pad64], axis=1)
    be2p = jnp.concatenate([be2.reshape(1, -1), pad64], axis=1)
    zpad = jnp.zeros((64, 128), jnp.float32)
    whp = jnp.concatenate(
        [jnp.concatenate([Wh1[:64], Wh1[64:]], axis=1), zpad], axis=0)
    whq = jnp.concatenate(
        [jnp.concatenate([Wh1[64:], Wh1[:64]], axis=1), zpad], axis=0)
    bh1p = jnp.concatenate([bh1.reshape(1, -1), pad64], axis=1)
    wh2p = jnp.concatenate([Wh2.reshape(1, -1), pad64], axis=1)

    degp = _deg_kernel()(dst_pad, ones128, z128)
    dinv, h1s = _tc_prep(degp, x, W1)
    acc1 = _scatter_kernel()(h1s, src_pad, dst_pad, z128)
    h2s = _tc_mid(acc1, h1s, dinv, b1r, g1r, be1r, w2p)
    acc2 = _scatter_kernel()(h2s, src_pad, dst_pad, z128)
    pq, qp = _tc_head(acc2, h2s, dinv, b2p, g2p, be2p, whp, whq)
    r, s = _pairs_kernel()(pq, qp, src, dst)
    out = _tc_final(r, s, bh1p, wh2p, bh2.reshape(1, 1))
    return out[:, 0]


# speedup vs baseline: 9.1416x; 1.3367x over previous
import functools
import math

import jax
import jax.numpy as jnp
from jax import lax
from jax.experimental import pallas as pl
from jax.experimental.pallas import tpu as pltpu
from jax.experimental.pallas import tpu_sc as plsc

_N = 10000
_NPAD = 10112
_E = 320000
_NC = 2
_NS = 16
_NW = _NC * _NS
_CHUNK = 128
_EC = 64
_EPAD = 327680
_ECHUNKS = _EPAD // (_NW * _CHUNK)
_GCHUNKS = _EPAD // (_NW * _EC)
_ROWS_PT = _NPAD // _NS
_P = 65536
_PCHUNKS = _P // (_NW * _CHUNK)
_BN_SCALE = 1.0 / math.sqrt(1.0 + 1e-5)

_mesh = plsc.VectorSubcoreMesh(core_axis_name="c", subcore_axis_name="s")


def _deg_kernel():
    @functools.partial(
        pl.kernel,
        out_type=jax.ShapeDtypeStruct((_NC, _NPAD, 128), jnp.float32),
        mesh=_mesh,
        scratch_types=[
            pltpu.VMEM((_CHUNK,), jnp.int32),
            pltpu.VMEM((_CHUNK,), jnp.int32),
            pltpu.VMEM((_CHUNK,), jnp.int32),
            pltpu.VMEM((_CHUNK,), jnp.int32),
            pltpu.VMEM((_CHUNK, 128), jnp.float32),
            pltpu.VMEM_SHARED((_NPAD, 128), jnp.float32),
            pltpu.SemaphoreType.DMA,
            pltpu.SemaphoreType.DMA,
            pltpu.SemaphoreType.DMA,
            pltpu.SemaphoreType.DMA,
        ],
    )
    def k(dst_hbm, ones_hbm, z_hbm, out_hbm, i0, i1, i2, i3, ones_v, acc_sh,
          s0, s1, s2, s3):
        c = lax.axis_index("c")
        s = lax.axis_index("s")
        wid = c * _NS + s
        row0 = s * _ROWS_PT
        base = wid * _ECHUNKS
        idxd = (i0, i1, i2, i3)
        sems = (s0, s1, s2, s3)
        pltpu.sync_copy(z_hbm, acc_sh.at[pl.ds(row0, _ROWS_PT)])
        pltpu.sync_copy(ones_hbm, ones_v)
        plsc.subcore_barrier()

        for j in range(4):
            off = pl.multiple_of((base + j) * _CHUNK, _CHUNK)
            pltpu.sync_copy(dst_hbm.at[pl.ds(off, _CHUNK)], idxd[j])
            pltpu.async_copy(ones_v, acc_sh.at[idxd[j]], sems[j], add=True)

        def body(i, carry):
            for j in range(4):
                chunk = i + j
                pltpu.make_async_copy(ones_v, acc_sh.at[idxd[j]],
                                      sems[j]).wait()

                @pl.when(chunk + 4 < _ECHUNKS)
                def _():
                    off = pl.multiple_of((base + chunk + 4) * _CHUNK, _CHUNK)
                    pltpu.sync_copy(dst_hbm.at[pl.ds(off, _CHUNK)], idxd[j])
                    pltpu.async_copy(ones_v, acc_sh.at[idxd[j]], sems[j],
                                     add=True)
            return carry

        lax.fori_loop(0, _ECHUNKS // 4, lambda i, cy: body(i * 4, cy), 0)
        plsc.subcore_barrier()
        pltpu.sync_copy(acc_sh.at[pl.ds(row0, _ROWS_PT)],
                        out_hbm.at[c].at[pl.ds(row0, _ROWS_PT)])

    return k


def _scatter_kernel():
    @functools.partial(
        pl.kernel,
        out_type=jax.ShapeDtypeStruct((_NC, _NPAD, 128), jnp.float32),
        mesh=_mesh,
        scratch_types=[
            pltpu.VMEM((_EC,), jnp.int32),
            pltpu.VMEM((_EC,), jnp.int32),
            pltpu.VMEM((_EC,), jnp.int32),
            pltpu.VMEM((_EC,), jnp.int32),
            pltpu.VMEM((_EC,), jnp.int32),
            pltpu.VMEM((_EC,), jnp.int32),
            pltpu.VMEM((_EC,), jnp.int32),
            pltpu.VMEM((_EC,), jnp.int32),
            pltpu.VMEM((_EC, 128), jnp.float32),
            pltpu.VMEM((_EC, 128), jnp.float32),
            pltpu.VMEM((_EC, 128), jnp.float32),
            pltpu.VMEM((_EC, 128), jnp.float32),
            pltpu.SemaphoreType.DMA,
            pltpu.SemaphoreType.DMA,
            pltpu.SemaphoreType.DMA,
            pltpu.SemaphoreType.DMA,
            pltpu.SemaphoreType.DMA,
            pltpu.SemaphoreType.DMA,
            pltpu.SemaphoreType.DMA,
            pltpu.SemaphoreType.DMA,
            pltpu.VMEM_SHARED((_NPAD, 128), jnp.float32),
        ],
    )
    def k(h_hbm, src_hbm, dst_hbm, z_hbm, out_hbm,
          is0, is1, is2, is3, id0, id1, id2, id3,
          r0, r1, r2, r3, g0, g1, g2, g3, ss0, ss1, ss2, ss3, acc_sh):
        idxs = (is0, is1, is2, is3)
        idxd = (id0, id1, id2, id3)
        rows = (r0, r1, r2, r3)
        gsem = (g0, g1, g2, g3)
        ssem = (ss0, ss1, ss2, ss3)
        c = lax.axis_index("c")
        s = lax.axis_index("s")
        wid = c * _NS + s
        row0 = s * _ROWS_PT
        base = wid * _GCHUNKS
        pltpu.sync_copy(z_hbm, acc_sh.at[pl.ds(row0, _ROWS_PT)])
        plsc.subcore_barrier()

        for j in range(4):
            off = pl.multiple_of((base + j) * _EC, _EC)
            pltpu.sync_copy(src_hbm.at[pl.ds(off, _EC)], idxs[j])
            pltpu.sync_copy(dst_hbm.at[pl.ds(off, _EC)], idxd[j])
            pltpu.async_copy(h_hbm.at[idxs[j]], rows[j], gsem[j])

        def body(i, carry):
            for j in range(4):
                chunk = i + j
                pltpu.make_async_copy(h_hbm.at[idxs[j]], rows[j],
                                      gsem[j]).wait()
                pltpu.async_copy(rows[j], acc_sh.at[idxd[j]], ssem[j],
                                 add=True)
                pltpu.make_async_copy(rows[j], acc_sh.at[idxd[j]],
                                      ssem[j]).wait()

                @pl.when(chunk + 4 < _GCHUNKS)
                def _():
                    off = pl.multiple_of((base + chunk + 4) * _EC, _EC)
                    pltpu.sync_copy(src_hbm.at[pl.ds(off, _EC)], idxs[j])
                    pltpu.sync_copy(dst_hbm.at[pl.ds(off, _EC)], idxd[j])
                    pltpu.async_copy(h_hbm.at[idxs[j]], rows[j], gsem[j])
            return carry

        lax.fori_loop(0, _GCHUNKS // 4, lambda i, cy: body(i * 4, cy), 0)
        plsc.subcore_barrier()
        pltpu.sync_copy(acc_sh.at[pl.ds(row0, _ROWS_PT)],
                        out_hbm.at[c].at[pl.ds(row0, _ROWS_PT)])

    return k


def _pairs_kernel():
    sds = jax.ShapeDtypeStruct((_P, 128), jnp.float32)

    @functools.partial(
        pl.kernel,
        out_type=(sds, sds),
        mesh=_mesh,
        scratch_types=[
            pltpu.VMEM((_PCHUNKS, _CHUNK), jnp.int32),
            pltpu.VMEM((_PCHUNKS, _CHUNK), jnp.int32),
            pltpu.VMEM((_CHUNK, 128), jnp.float32),
            pltpu.VMEM((_CHUNK, 128), jnp.float32),
            pltpu.VMEM((_CHUNK, 128), jnp.float32),
            pltpu.VMEM((_CHUNK, 128), jnp.float32),
            pltpu.SemaphoreType.DMA,
            pltpu.SemaphoreType.DMA,
            pltpu.SemaphoreType.DMA,
            pltpu.SemaphoreType.DMA,
            pltpu.SemaphoreType.DMA,
            pltpu.SemaphoreType.DMA,
            pltpu.SemaphoreType.DMA,
            pltpu.SemaphoreType.DMA,
        ],
    )
    def k(pq_hbm, qp_hbm, src_hbm, dst_hbm, outr_hbm, outs_hbm,
          src_v, dst_v, bp0, bp1, bq0, bq1,
          gs0, gs1, gs2, gs3, ws0, ws1, ws2, ws3):
        bufp = (bp0, bp1)
        bufq = (bq0, bq1)
        gsem = (gs0, gs1, gs2, gs3)
        wsem = (ws0, ws1, ws2, ws3)
        c = lax.axis_index("c")
        s = lax.axis_index("s")
        wid = c * _NS + s
        base = wid * _PCHUNKS
        pltpu.sync_copy(src_hbm.at[pl.ds(base, _PCHUNKS)], src_v)
        pltpu.sync_copy(dst_hbm.at[pl.ds(base, _PCHUNKS)], dst_v)
        for j in range(2):
            pltpu.async_copy(pq_hbm.at[src_v.at[j]], bufp[j], gsem[j])
            pltpu.async_copy(qp_hbm.at[dst_v.at[j]], bufq[j], gsem[2 + j])

        def body(i, carry):
            for j in range(2):
                chunk = i + j
                off = pl.multiple_of((base + chunk) * _CHUNK, _CHUNK)
                pltpu.make_async_copy(pq_hbm.at[src_v.at[chunk]], bufp[j],
                                      gsem[j]).wait()
                pltpu.make_async_copy(qp_hbm.at[dst_v.at[chunk]], bufq[j],
                                      gsem[2 + j]).wait()
                pltpu.async_copy(bufp[j], outr_hbm.at[pl.ds(off, _CHUNK)],
                                 wsem[j])
                pltpu.async_copy(bufq[j], outs_hbm.at[pl.ds(off, _CHUNK)],
                                 wsem[2 + j])
                pltpu.make_async_copy(bufp[j], outr_hbm.at[pl.ds(off, _CHUNK)],
                                      wsem[j]).wait()
                pltpu.make_async_copy(bufq[j], outs_hbm.at[pl.ds(off, _CHUNK)],
                                      wsem[2 + j]).wait()

                @pl.when(chunk + 2 < _PCHUNKS)
                def _():
                    pltpu.async_copy(pq_hbm.at[src_v.at[chunk + 2]], bufp[j],
                                     gsem[j])
                    pltpu.async_copy(qp_hbm.at[dst_v.at[chunk + 2]], bufq[j],
                                     gsem[2 + j])
            return carry

        lax.fori_loop(0, _PCHUNKS // 2, lambda i, cy: body(i * 2, cy), 0)

    return k


def _tc_prep(degp, x, w1):
    def body(deg_ref, x_ref, w_ref, dinv_ref, h1s_ref):
        dinv = lax.rsqrt(deg_ref[0] + deg_ref[1] + 1.0)
        dinv_ref[...] = dinv
        h = jnp.dot(x_ref[...], w_ref[...],
                    preferred_element_type=jnp.float32)
        h1s_ref[...] = dinv[:_N] * h

    return pl.pallas_call(
        body,
        out_shape=(jax.ShapeDtypeStruct((_NPAD, 128), jnp.float32),
                   jax.ShapeDtypeStruct((_N, 128), jnp.float32)),
    )(degp, x, w1)


def _tc_mid(acc1, h1s, dinv, b1, g1, be1, w2p):
    def body(acc_ref, h_ref, dinv_ref, b_ref, g_ref, be_ref, w_ref, out_ref):
        dv = dinv_ref[:_N]
        o1 = dv * (acc_ref[0, :_N] + acc_ref[1, :_N] + h_ref[...]) + b_ref[...]
        z1 = jnp.maximum(g_ref[...] * (o1 * _BN_SCALE) + be_ref[...], 0.0)
        out_ref[...] = dv * jnp.dot(z1, w_ref[...],
                                    preferred_element_type=jnp.float32)

    return pl.pallas_call(
        body,
        out_shape=jax.ShapeDtypeStruct((_N, 128), jnp.float32),
    )(acc1, h1s, dinv, b1, g1, be1, w2p)


def _tc_head(acc2, h2s, dinv, b2, g2, be2, whp, whq):
    def body(acc_ref, h_ref, dinv_ref, b_ref, g_ref, be_ref, wp_ref, wq_ref,
             pq_ref, qp_ref):
        dv = dinv_ref[:_N]
        o2 = dv * (acc_ref[0, :_N] + acc_ref[1, :_N] + h_ref[...]) + b_ref[...]
        z2 = jnp.maximum(g_ref[...] * (o2 * _BN_SCALE) + be_ref[...], 0.0)
        pq_ref[...] = jnp.dot(z2, wp_ref[...],
                              preferred_element_type=jnp.float32)
        qp_ref[...] = jnp.dot(z2, wq_ref[...],
                              preferred_element_type=jnp.float32)

    return pl.pallas_call(
        body,
        out_shape=(jax.ShapeDtypeStruct((_N, 128), jnp.float32),
                   jax.ShapeDtypeStruct((_N, 128), jnp.float32)),
    )(acc2, h2s, dinv, b2, g2, be2, whp, whq)


def _tc_final(r, s, bh1p, wh2p, bh2):
    blk = 8192

    def body(r_ref, s_ref, b_ref, w_ref, b2_ref, out_ref):
        h = jnp.maximum(r_ref[...] + s_ref[...] + b_ref[...], 0.0)
        acc = jnp.sum(h * w_ref[...], axis=1, keepdims=True) + b2_ref[0, 0]
        out_ref[...] = 1.0 / (1.0 + jnp.exp(-acc))

    grid = _P // blk
    return pl.pallas_call(
        body,
        grid=(grid,),
        in_specs=[
            pl.BlockSpec((blk, 128), lambda i: (i, 0)),
            pl.BlockSpec((blk, 128), lambda i: (i, 0)),
            pl.BlockSpec((1, 128), lambda i: (0, 0)),
            pl.BlockSpec((1, 128), lambda i: (0, 0)),
            pl.BlockSpec((1, 1), lambda i: (0, 0)),
        ],
        out_specs=pl.BlockSpec((blk, 1), lambda i: (i, 0)),
        out_shape=jax.ShapeDtypeStruct((_P, 1), jnp.float32),
    )(r, s, bh1p, wh2p, bh2)


def kernel(x, ei, src, dst, W1, b1, g1, be1, W2, b2, g2, be2,
           Wh1, bh1, Wh2, bh2):
    npd = _EPAD - _E
    src_pad = jnp.concatenate([ei[0], jnp.zeros((npd,), jnp.int32)])
    dst_pad = jnp.concatenate([ei[1], jnp.full((npd,), _N, jnp.int32)])
    src2d = src.reshape(-1, _CHUNK)
    dst2d = dst.reshape(-1, _CHUNK)

    ones128 = jnp.ones((_CHUNK, 128), jnp.float32)
    z128 = jnp.zeros((_ROWS_PT, 128), jnp.float32)

    w2p = jnp.concatenate([W2, jnp.zeros((128, 64), jnp.float32)], axis=1)
    b1r, g1r, be1r = (a.reshape(1, -1) for a in (b1, g1, be1))
    pad64 = jnp.zeros((1, 64), jnp.float32)
    b2p = jnp.concatenate([b2.reshape(1, -1), pad64], axis=1)
    g2p = jnp.concatenate([g2.reshape(1, -1), pad64], axis=1)
    be2p = jnp.concatenate([be2.reshape(1, -1), pad64], axis=1)
    zpad = jnp.zeros((64, 128), jnp.float32)
    whp = jnp.concatenate(
        [jnp.concatenate([Wh1[:64], Wh1[64:]], axis=1), zpad], axis=0)
    whq = jnp.concatenate(
        [jnp.concatenate([Wh1[64:], Wh1[:64]], axis=1), zpad], axis=0)
    bh1p = jnp.concatenate([bh1.reshape(1, -1), pad64], axis=1)
    wh2p = jnp.concatenate([Wh2.reshape(1, -1), pad64], axis=1)

    degp = _deg_kernel()(dst_pad, ones128, z128)
    dinv, h1s = _tc_prep(degp, x, W1)
    acc1 = _scatter_kernel()(h1s, src_pad, dst_pad, z128)
    h2s = _tc_mid(acc1, h1s, dinv, b1r, g1r, be1r, w2p)
    acc2 = _scatter_kernel()(h2s, src_pad, dst_pad, z128)
    pq, qp = _tc_head(acc2, h2s, dinv, b2p, g2p, be2p, whp, whq)
    r, s = _pairs_kernel()(pq, qp, src2d, dst2d)
    out = _tc_final(r, s, bh1p, wh2p, bh2.reshape(1, 1))
    return out[:, 0]


# speedup vs baseline: 9.8802x; 1.0808x over previous
import functools
import math

import jax
import jax.numpy as jnp
from jax import lax
from jax.experimental import pallas as pl
from jax.experimental.pallas import tpu as pltpu
from jax.experimental.pallas import tpu_sc as plsc

_N = 10000
_NPAD = 10112
_E = 320000
_NC = 2
_NS = 16
_NW = _NC * _NS
_CHUNK = 128
_EC = 64
_EPAD = 327680
_ECHUNKS = _EPAD // (_NW * _CHUNK)
_N0 = 210
_N1 = 110
_NSLOTS = 5
_ROWS_PT = _NPAD // _NS
_P = 65536
_PCHUNKS = _P // (_NW * _CHUNK)
_BN_SCALE = 1.0 / math.sqrt(1.0 + 1e-5)

_mesh = plsc.VectorSubcoreMesh(core_axis_name="c", subcore_axis_name="s")


def _deg_kernel():
    @functools.partial(
        pl.kernel,
        out_type=jax.ShapeDtypeStruct((_NC, _NPAD, 128), jnp.float32),
        mesh=_mesh,
        scratch_types=[
            pltpu.VMEM((_CHUNK,), jnp.int32),
            pltpu.VMEM((_CHUNK,), jnp.int32),
            pltpu.VMEM((_CHUNK,), jnp.int32),
            pltpu.VMEM((_CHUNK,), jnp.int32),
            pltpu.VMEM((_CHUNK, 128), jnp.float32),
            pltpu.VMEM_SHARED((_NPAD, 128), jnp.float32),
            pltpu.SemaphoreType.DMA,
            pltpu.SemaphoreType.DMA,
            pltpu.SemaphoreType.DMA,
            pltpu.SemaphoreType.DMA,
        ],
    )
    def k(dst_hbm, ones_hbm, z_hbm, out_hbm, i0, i1, i2, i3, ones_v, acc_sh,
          s0, s1, s2, s3):
        c = lax.axis_index("c")
        s = lax.axis_index("s")
        wid = c * _NS + s
        row0 = s * _ROWS_PT
        base = wid * _ECHUNKS
        idxd = (i0, i1, i2, i3)
        sems = (s0, s1, s2, s3)
        pltpu.sync_copy(z_hbm, acc_sh.at[pl.ds(row0, _ROWS_PT)])
        pltpu.sync_copy(ones_hbm, ones_v)
        plsc.subcore_barrier()

        for j in range(4):
            off = pl.multiple_of((base + j) * _CHUNK, _CHUNK)
            pltpu.sync_copy(dst_hbm.at[pl.ds(off, _CHUNK)], idxd[j])
            pltpu.async_copy(ones_v, acc_sh.at[idxd[j]], sems[j], add=True)

        def body(i, carry):
            for j in range(4):
                chunk = i + j
                pltpu.make_async_copy(ones_v, acc_sh.at[idxd[j]],
                                      sems[j]).wait()

                @pl.when(chunk + 4 < _ECHUNKS)
                def _():
                    off = pl.multiple_of((base + chunk + 4) * _CHUNK, _CHUNK)
                    pltpu.sync_copy(dst_hbm.at[pl.ds(off, _CHUNK)], idxd[j])
                    pltpu.async_copy(ones_v, acc_sh.at[idxd[j]], sems[j],
                                     add=True)
            return carry

        lax.fori_loop(0, _ECHUNKS // 4, lambda i, cy: body(i * 4, cy), 0)
        plsc.subcore_barrier()
        pltpu.sync_copy(acc_sh.at[pl.ds(row0, _ROWS_PT)],
                        out_hbm.at[c].at[pl.ds(row0, _ROWS_PT)])

    return k


def _scatter_kernel():
    ns = _NSLOTS
    scratch = (
        [pltpu.VMEM((_EC,), jnp.int32) for _ in range(2 * ns)]
        + [pltpu.VMEM((_EC, 128), jnp.float32) for _ in range(ns)]
        + [pltpu.SemaphoreType.DMA for _ in range(2 * ns)]
        + [pltpu.VMEM_SHARED((_NPAD, 128), jnp.float32)]
    )

    @functools.partial(
        pl.kernel,
        out_type=jax.ShapeDtypeStruct((_NC, _NPAD, 128), jnp.float32),
        mesh=_mesh,
        scratch_types=scratch,
    )
    def k(h_hbm, src_hbm, dst_hbm, z_hbm, out_hbm, *scr):
        idxs = scr[:ns]
        idxd = scr[ns:2 * ns]
        rows = scr[2 * ns:3 * ns]
        gsem = scr[3 * ns:4 * ns]
        ssem = scr[4 * ns:5 * ns]
        acc_sh = scr[5 * ns]
        c = lax.axis_index("c")
        s = lax.axis_index("s")
        row0 = s * _ROWS_PT
        nloc = jnp.where(c == 0, _N0, _N1)
        base = jnp.where(c == 0, s * _N0, _NS * _N0 + s * _N1)
        pltpu.sync_copy(z_hbm, acc_sh.at[pl.ds(row0, _ROWS_PT)])
        plsc.subcore_barrier()

        for j in range(ns):
            off = pl.multiple_of((base + j) * _EC, _EC)
            pltpu.sync_copy(src_hbm.at[pl.ds(off, _EC)], idxs[j])
            pltpu.sync_copy(dst_hbm.at[pl.ds(off, _EC)], idxd[j])
            pltpu.async_copy(h_hbm.at[idxs[j]], rows[j], gsem[j])

        def body(i, carry):
            for j in range(ns):
                chunk = i + j
                pltpu.make_async_copy(h_hbm.at[idxs[j]], rows[j],
                                      gsem[j]).wait()
                pltpu.async_copy(rows[j], acc_sh.at[idxd[j]], ssem[j],
                                 add=True)
                pltpu.make_async_copy(rows[j], acc_sh.at[idxd[j]],
                                      ssem[j]).wait()

                @pl.when(chunk + ns < nloc)
                def _():
                    off = pl.multiple_of((base + chunk + ns) * _EC, _EC)
                    pltpu.sync_copy(src_hbm.at[pl.ds(off, _EC)], idxs[j])
                    pltpu.sync_copy(dst_hbm.at[pl.ds(off, _EC)], idxd[j])
                    pltpu.async_copy(h_hbm.at[idxs[j]], rows[j], gsem[j])
            return carry

        lax.fori_loop(0, nloc // ns, lambda i, cy: body(i * ns, cy), 0)
        plsc.subcore_barrier()
        pltpu.sync_copy(acc_sh.at[pl.ds(row0, _ROWS_PT)],
                        out_hbm.at[c].at[pl.ds(row0, _ROWS_PT)])

    return k


def _pairs_kernel():
    sds = jax.ShapeDtypeStruct((_P, 128), jnp.float32)

    @functools.partial(
        pl.kernel,
        out_type=(sds, sds),
        mesh=_mesh,
        scratch_types=[
            pltpu.VMEM((_PCHUNKS, _CHUNK), jnp.int32),
            pltpu.VMEM((_PCHUNKS, _CHUNK), jnp.int32),
            pltpu.VMEM((_CHUNK, 128), jnp.float32),
            pltpu.VMEM((_CHUNK, 128), jnp.float32),
            pltpu.VMEM((_CHUNK, 128), jnp.float32),
            pltpu.VMEM((_CHUNK, 128), jnp.float32),
            pltpu.SemaphoreType.DMA,
            pltpu.SemaphoreType.DMA,
            pltpu.SemaphoreType.DMA,
            pltpu.SemaphoreType.DMA,
            pltpu.SemaphoreType.DMA,
            pltpu.SemaphoreType.DMA,
            pltpu.SemaphoreType.DMA,
            pltpu.SemaphoreType.DMA,
        ],
    )
    def k(pq_hbm, qp_hbm, src_hbm, dst_hbm, outr_hbm, outs_hbm,
          src_v, dst_v, bp0, bp1, bq0, bq1,
          gs0, gs1, gs2, gs3, ws0, ws1, ws2, ws3):
        bufp = (bp0, bp1)
        bufq = (bq0, bq1)
        gsem = (gs0, gs1, gs2, gs3)
        wsem = (ws0, ws1, ws2, ws3)
        c = lax.axis_index("c")
        s = lax.axis_index("s")
        wid = c * _NS + s
        base = wid * _PCHUNKS
        pltpu.sync_copy(src_hbm.at[pl.ds(base, _PCHUNKS)], src_v)
        pltpu.sync_copy(dst_hbm.at[pl.ds(base, _PCHUNKS)], dst_v)
        for j in range(2):
            pltpu.async_copy(pq_hbm.at[src_v.at[j]], bufp[j], gsem[j])
            pltpu.async_copy(qp_hbm.at[dst_v.at[j]], bufq[j], gsem[2 + j])

        def body(i, carry):
            for j in range(2):
                chunk = i + j
                off = pl.multiple_of((base + chunk) * _CHUNK, _CHUNK)
                pltpu.make_async_copy(pq_hbm.at[src_v.at[chunk]], bufp[j],
                                      gsem[j]).wait()
                pltpu.make_async_copy(qp_hbm.at[dst_v.at[chunk]], bufq[j],
                                      gsem[2 + j]).wait()
                pltpu.async_copy(bufp[j], outr_hbm.at[pl.ds(off, _CHUNK)],
                                 wsem[j])
                pltpu.async_copy(bufq[j], outs_hbm.at[pl.ds(off, _CHUNK)],
                                 wsem[2 + j])
                pltpu.make_async_copy(bufp[j], outr_hbm.at[pl.ds(off, _CHUNK)],
                                      wsem[j]).wait()
                pltpu.make_async_copy(bufq[j], outs_hbm.at[pl.ds(off, _CHUNK)],
                                      wsem[2 + j]).wait()

                @pl.when(chunk + 2 < _PCHUNKS)
                def _():
                    pltpu.async_copy(pq_hbm.at[src_v.at[chunk + 2]], bufp[j],
                                     gsem[j])
                    pltpu.async_copy(qp_hbm.at[dst_v.at[chunk + 2]], bufq[j],
                                     gsem[2 + j])
            return carry

        lax.fori_loop(0, _PCHUNKS // 2, lambda i, cy: body(i * 2, cy), 0)

    return k


def _tc_prep(degp, x, w1):
    def body(deg_ref, x_ref, w_ref, dinv_ref, h1s_ref):
        dinv = lax.rsqrt(deg_ref[0] + deg_ref[1] + 1.0)
        dinv_ref[...] = dinv
        h = jnp.dot(x_ref[...], w_ref[...],
                    preferred_element_type=jnp.float32)
        h1s_ref[...] = dinv[:_N] * h

    return pl.pallas_call(
        body,
        out_shape=(jax.ShapeDtypeStruct((_NPAD, 128), jnp.float32),
                   jax.ShapeDtypeStruct((_N, 128), jnp.float32)),
    )(degp, x, w1)


def _tc_mid(acc1, h1s, dinv, b1, g1, be1, w2p):
    def body(acc_ref, h_ref, dinv_ref, b_ref, g_ref, be_ref, w_ref, out_ref):
        dv = dinv_ref[:_N]
        o1 = dv * (acc_ref[0, :_N] + acc_ref[1, :_N] + h_ref[...]) + b_ref[...]
        z1 = jnp.maximum(g_ref[...] * (o1 * _BN_SCALE) + be_ref[...], 0.0)
        out_ref[...] = dv * jnp.dot(z1, w_ref[...],
                                    preferred_element_type=jnp.float32)

    return pl.pallas_call(
        body,
        out_shape=jax.ShapeDtypeStruct((_N, 128), jnp.float32),
    )(acc1, h1s, dinv, b1, g1, be1, w2p)


def _tc_head(acc2, h2s, dinv, b2, g2, be2, whp, whq):
    def body(acc_ref, h_ref, dinv_ref, b_ref, g_ref, be_ref, wp_ref, wq_ref,
             pq_ref, qp_ref):
        dv = dinv_ref[:_N]
        o2 = dv * (acc_ref[0, :_N] + acc_ref[1, :_N] + h_ref[...]) + b_ref[...]
        z2 = jnp.maximum(g_ref[...] * (o2 * _BN_SCALE) + be_ref[...], 0.0)
        pq_ref[...] = jnp.dot(z2, wp_ref[...],
                              preferred_element_type=jnp.float32)
        qp_ref[...] = jnp.dot(z2, wq_ref[...],
                              preferred_element_type=jnp.float32)

    return pl.pallas_call(
        body,
        out_shape=(jax.ShapeDtypeStruct((_N, 128), jnp.float32),
                   jax.ShapeDtypeStruct((_N, 128), jnp.float32)),
    )(acc2, h2s, dinv, b2, g2, be2, whp, whq)


def _tc_final(r, s, bh1p, wh2p, bh2):
    blk = 8192

    def body(r_ref, s_ref, b_ref, w_ref, b2_ref, out_ref):
        h = jnp.maximum(r_ref[...] + s_ref[...] + b_ref[...], 0.0)
        acc = jnp.sum(h * w_ref[...], axis=1, keepdims=True) + b2_ref[0, 0]
        out_ref[...] = 1.0 / (1.0 + jnp.exp(-acc))

    grid = _P // blk
    return pl.pallas_call(
        body,
        grid=(grid,),
        in_specs=[
            pl.BlockSpec((blk, 128), lambda i: (i, 0)),
            pl.BlockSpec((blk, 128), lambda i: (i, 0)),
            pl.BlockSpec((1, 128), lambda i: (0, 0)),
            pl.BlockSpec((1, 128), lambda i: (0, 0)),
            pl.BlockSpec((1, 1), lambda i: (0, 0)),
        ],
        out_specs=pl.BlockSpec((blk, 1), lambda i: (i, 0)),
        out_shape=jax.ShapeDtypeStruct((_P, 1), jnp.float32),
    )(r, s, bh1p, wh2p, bh2)


def kernel(x, ei, src, dst, W1, b1, g1, be1, W2, b2, g2, be2,
           Wh1, bh1, Wh2, bh2):
    npd = _EPAD - _E
    src_pad = jnp.concatenate([ei[0], jnp.zeros((npd,), jnp.int32)])
    dst_pad = jnp.concatenate([ei[1], jnp.full((npd,), _N, jnp.int32)])
    src2d = src.reshape(-1, _CHUNK)
    dst2d = dst.reshape(-1, _CHUNK)

    ones128 = jnp.ones((_CHUNK, 128), jnp.float32)
    z128 = jnp.zeros((_ROWS_PT, 128), jnp.float32)

    w2p = jnp.concatenate([W2, jnp.zeros((128, 64), jnp.float32)], axis=1)
    b1r, g1r, be1r = (a.reshape(1, -1) for a in (b1, g1, be1))
    pad64 = jnp.zeros((1, 64), jnp.float32)
    b2p = jnp.concatenate([b2.reshape(1, -1), pad64], axis=1)
    g2p = jnp.concatenate([g2.reshape(1, -1), pad64], axis=1)
    be2p = jnp.concatenate([be2.reshape(1, -1), pad64], axis=1)
    zpad = jnp.zeros((64, 128), jnp.float32)
    whp = jnp.concatenate(
        [jnp.concatenate([Wh1[:64], Wh1[64:]], axis=1), zpad], axis=0)
    whq = jnp.concatenate(
        [jnp.concatenate([Wh1[64:], Wh1[:64]], axis=1), zpad], axis=0)
    bh1p = jnp.concatenate([bh1.reshape(1, -1), pad64], axis=1)
    wh2p = jnp.concatenate([Wh2.reshape(1, -1), pad64], axis=1)

    degp = _deg_kernel()(dst_pad, ones128, z128)
    dinv, h1s = _tc_prep(degp, x, W1)
    acc1 = _scatter_kernel()(h1s, src_pad, dst_pad, z128)
    h2s = _tc_mid(acc1, h1s, dinv, b1r, g1r, be1r, w2p)
    acc2 = _scatter_kernel()(h2s, src_pad, dst_pad, z128)
    pq, qp = _tc_head(acc2, h2s, dinv, b2p, g2p, be2p, whp, whq)
    r, s = _pairs_kernel()(pq, qp, src2d, dst2d)
    out = _tc_final(r, s, bh1p, wh2p, bh2.reshape(1, 1))
    return out[:, 0]


# speedup vs baseline: 15.4508x; 1.5638x over previous
import functools
import math

import jax
import jax.numpy as jnp
from jax import lax
from jax.experimental import pallas as pl
from jax.experimental.pallas import tpu as pltpu
from jax.experimental.pallas import tpu_sc as plsc

_N = 10000
_NPAD = 10112
_E = 320000
_NC = 2
_NS = 16
_NW = _NC * _NS
_CHUNK = 128
_EC = 64
_EPAD = 327680
_ECHUNKS = _EPAD // (_NW * _CHUNK)
_N0 = 160
_N1 = 160
_NSLOTS = 5
_ROWS_PT = _NPAD // _NS
_P = 65536
_PCHUNKS = _P // (_NW * _CHUNK)
_BN_SCALE = 1.0 / math.sqrt(1.0 + 1e-5)

_mesh = plsc.VectorSubcoreMesh(core_axis_name="c", subcore_axis_name="s")


def _deg_kernel():
    @functools.partial(
        pl.kernel,
        out_type=jax.ShapeDtypeStruct((_NC, _NPAD, 128), jnp.float32),
        mesh=_mesh,
        scratch_types=[
            pltpu.VMEM((_CHUNK,), jnp.int32),
            pltpu.VMEM((_CHUNK,), jnp.int32),
            pltpu.VMEM((_CHUNK,), jnp.int32),
            pltpu.VMEM((_CHUNK,), jnp.int32),
            pltpu.VMEM((_CHUNK, 128), jnp.float32),
            pltpu.VMEM_SHARED((_NPAD, 128), jnp.float32),
            pltpu.SemaphoreType.DMA,
            pltpu.SemaphoreType.DMA,
            pltpu.SemaphoreType.DMA,
            pltpu.SemaphoreType.DMA,
        ],
    )
    def k(dst_hbm, ones_hbm, z_hbm, out_hbm, i0, i1, i2, i3, ones_v, acc_sh,
          s0, s1, s2, s3):
        c = lax.axis_index("c")
        s = lax.axis_index("s")
        wid = c * _NS + s
        row0 = s * _ROWS_PT
        base = wid * _ECHUNKS
        idxd = (i0, i1, i2, i3)
        sems = (s0, s1, s2, s3)
        pltpu.sync_copy(z_hbm, acc_sh.at[pl.ds(row0, _ROWS_PT)])
        pltpu.sync_copy(ones_hbm, ones_v)
        plsc.subcore_barrier()

        for j in range(4):
            off = pl.multiple_of((base + j) * _CHUNK, _CHUNK)
            pltpu.sync_copy(dst_hbm.at[pl.ds(off, _CHUNK)], idxd[j])
            pltpu.async_copy(ones_v, acc_sh.at[idxd[j]], sems[j], add=True)

        def body(i, carry):
            for j in range(4):
                chunk = i + j
                pltpu.make_async_copy(ones_v, acc_sh.at[idxd[j]],
                                      sems[j]).wait()

                @pl.when(chunk + 4 < _ECHUNKS)
                def _():
                    off = pl.multiple_of((base + chunk + 4) * _CHUNK, _CHUNK)
                    pltpu.sync_copy(dst_hbm.at[pl.ds(off, _CHUNK)], idxd[j])
                    pltpu.async_copy(ones_v, acc_sh.at[idxd[j]], sems[j],
                                     add=True)
            return carry

        lax.fori_loop(0, _ECHUNKS // 4, lambda i, cy: body(i * 4, cy), 0)
        plsc.subcore_barrier()
        pltpu.sync_copy(acc_sh.at[pl.ds(row0, _ROWS_PT)],
                        out_hbm.at[c].at[pl.ds(row0, _ROWS_PT)])

    return k


def _scatter_kernel():
    ns = _NSLOTS
    scratch = (
        [pltpu.VMEM((_EC,), jnp.int32) for _ in range(2 * ns)]
        + [pltpu.VMEM((_EC, 128), jnp.float32) for _ in range(ns)]
        + [pltpu.SemaphoreType.DMA for _ in range(2 * ns)]
        + [pltpu.VMEM_SHARED((_NPAD, 128), jnp.float32)]
    )

    @functools.partial(
        pl.kernel,
        out_type=jax.ShapeDtypeStruct((_NC, _NPAD, 128), jnp.float32),
        mesh=_mesh,
        scratch_types=scratch,
    )
    def k(h_hbm, src_hbm, dst_hbm, z_hbm, out_hbm, *scr):
        idxs = scr[:ns]
        idxd = scr[ns:2 * ns]
        rows = scr[2 * ns:3 * ns]
        gsem = scr[3 * ns:4 * ns]
        ssem = scr[4 * ns:5 * ns]
        acc_sh = scr[5 * ns]
        c = lax.axis_index("c")
        s = lax.axis_index("s")
        row0 = s * _ROWS_PT
        nloc = jnp.where(c == 0, _N0, _N1)
        base = jnp.where(c == 0, s * _N0, _NS * _N0 + s * _N1)
        pltpu.sync_copy(z_hbm, acc_sh.at[pl.ds(row0, _ROWS_PT)])
        plsc.subcore_barrier()

        for j in range(ns):
            off = pl.multiple_of((base + j) * _EC, _EC)
            pltpu.sync_copy(src_hbm.at[pl.ds(off, _EC)], idxs[j])
            pltpu.sync_copy(dst_hbm.at[pl.ds(off, _EC)], idxd[j])
            pltpu.async_copy(h_hbm.at[idxs[j]], rows[j], gsem[j])

        def body(i, carry):
            for j in range(ns):
                chunk = i + j
                pltpu.make_async_copy(h_hbm.at[idxs[j]], rows[j],
                                      gsem[j]).wait()
                pltpu.async_copy(rows[j], acc_sh.at[idxd[j]], ssem[j],
                                 add=True)
                pltpu.make_async_copy(rows[j], acc_sh.at[idxd[j]],
                                      ssem[j]).wait()

                @pl.when(chunk + ns < nloc)
                def _():
                    off = pl.multiple_of((base + chunk + ns) * _EC, _EC)
                    pltpu.sync_copy(src_hbm.at[pl.ds(off, _EC)], idxs[j])
                    pltpu.sync_copy(dst_hbm.at[pl.ds(off, _EC)], idxd[j])
                    pltpu.async_copy(h_hbm.at[idxs[j]], rows[j], gsem[j])
            return carry

        lax.fori_loop(0, nloc // ns, lambda i, cy: body(i * ns, cy), 0)
        plsc.subcore_barrier()
        pltpu.sync_copy(acc_sh.at[pl.ds(row0, _ROWS_PT)],
                        out_hbm.at[c].at[pl.ds(row0, _ROWS_PT)])

    return k


def _pairs_kernel():
    sds = jax.ShapeDtypeStruct((_P, 128), jnp.float32)

    @functools.partial(
        pl.kernel,
        out_type=(sds, sds),
        mesh=_mesh,
        scratch_types=[
            pltpu.VMEM((_PCHUNKS, _CHUNK), jnp.int32),
            pltpu.VMEM((_PCHUNKS, _CHUNK), jnp.int32),
            pltpu.VMEM((_CHUNK, 128), jnp.float32),
            pltpu.VMEM((_CHUNK, 128), jnp.float32),
            pltpu.VMEM((_CHUNK, 128), jnp.float32),
            pltpu.VMEM((_CHUNK, 128), jnp.float32),
            pltpu.SemaphoreType.DMA,
            pltpu.SemaphoreType.DMA,
            pltpu.SemaphoreType.DMA,
            pltpu.SemaphoreType.DMA,
            pltpu.SemaphoreType.DMA,
            pltpu.SemaphoreType.DMA,
            pltpu.SemaphoreType.DMA,
            pltpu.SemaphoreType.DMA,
        ],
    )
    def k(pq_hbm, qp_hbm, src_hbm, dst_hbm, outr_hbm, outs_hbm,
          src_v, dst_v, bp0, bp1, bq0, bq1,
          gs0, gs1, gs2, gs3, ws0, ws1, ws2, ws3):
        bufp = (bp0, bp1)
        bufq = (bq0, bq1)
        gsem = (gs0, gs1, gs2, gs3)
        wsem = (ws0, ws1, ws2, ws3)
        c = lax.axis_index("c")
        s = lax.axis_index("s")
        wid = c * _NS + s
        base = wid * _PCHUNKS
        pltpu.sync_copy(src_hbm.at[pl.ds(base, _PCHUNKS)], src_v)
        pltpu.sync_copy(dst_hbm.at[pl.ds(base, _PCHUNKS)], dst_v)
        for j in range(2):
            pltpu.async_copy(pq_hbm.at[src_v.at[j]], bufp[j], gsem[j])
            pltpu.async_copy(qp_hbm.at[dst_v.at[j]], bufq[j], gsem[2 + j])

        def body(i, carry):
            for j in range(2):
                chunk = i + j
                off = pl.multiple_of((base + chunk) * _CHUNK, _CHUNK)
                pltpu.make_async_copy(pq_hbm.at[src_v.at[chunk]], bufp[j],
                                      gsem[j]).wait()
                pltpu.make_async_copy(qp_hbm.at[dst_v.at[chunk]], bufq[j],
                                      gsem[2 + j]).wait()
                pltpu.async_copy(bufp[j], outr_hbm.at[pl.ds(off, _CHUNK)],
                                 wsem[j])
                pltpu.async_copy(bufq[j], outs_hbm.at[pl.ds(off, _CHUNK)],
                                 wsem[2 + j])
                pltpu.make_async_copy(bufp[j], outr_hbm.at[pl.ds(off, _CHUNK)],
                                      wsem[j]).wait()
                pltpu.make_async_copy(bufq[j], outs_hbm.at[pl.ds(off, _CHUNK)],
                                      wsem[2 + j]).wait()

                @pl.when(chunk + 2 < _PCHUNKS)
                def _():
                    pltpu.async_copy(pq_hbm.at[src_v.at[chunk + 2]], bufp[j],
                                     gsem[j])
                    pltpu.async_copy(qp_hbm.at[dst_v.at[chunk + 2]], bufq[j],
                                     gsem[2 + j])
            return carry

        lax.fori_loop(0, _PCHUNKS // 2, lambda i, cy: body(i * 2, cy), 0)

    return k


def _tc_prep(degp, x, w1):
    def body(deg_ref, x_ref, w_ref, dinv_ref, h1s_ref):
        dinv = lax.rsqrt(deg_ref[0] + deg_ref[1] + 1.0)
        dinv_ref[...] = dinv
        h = jnp.dot(x_ref[...], w_ref[...],
                    preferred_element_type=jnp.float32)
        h1s_ref[...] = dinv[:_N] * h

    return pl.pallas_call(
        body,
        out_shape=(jax.ShapeDtypeStruct((_NPAD, 128), jnp.float32),
                   jax.ShapeDtypeStruct((_N, 128), jnp.float32)),
    )(degp, x, w1)


def _tc_mid(acc1, h1s, dinv, b1, g1, be1, w2p):
    def body(acc_ref, h_ref, dinv_ref, b_ref, g_ref, be_ref, w_ref, out_ref):
        dv = dinv_ref[:_N]
        o1 = dv * (acc_ref[0, :_N] + acc_ref[1, :_N] + h_ref[...]) + b_ref[...]
        z1 = jnp.maximum(g_ref[...] * (o1 * _BN_SCALE) + be_ref[...], 0.0)
        out_ref[...] = dv * jnp.dot(z1, w_ref[...],
                                    preferred_element_type=jnp.float32)

    return pl.pallas_call(
        body,
        out_shape=jax.ShapeDtypeStruct((_N, 128), jnp.float32),
    )(acc1, h1s, dinv, b1, g1, be1, w2p)


def _tc_head(acc2, h2s, dinv, b2, g2, be2, whp, whq):
    def body(acc_ref, h_ref, dinv_ref, b_ref, g_ref, be_ref, wp_ref, wq_ref,
             pq_ref, qp_ref):
        dv = dinv_ref[:_N]
        o2 = dv * (acc_ref[0, :_N] + acc_ref[1, :_N] + h_ref[...]) + b_ref[...]
        z2 = jnp.maximum(g_ref[...] * (o2 * _BN_SCALE) + be_ref[...], 0.0)
        pq_ref[...] = jnp.dot(z2, wp_ref[...],
                              preferred_element_type=jnp.float32)
        qp_ref[...] = jnp.dot(z2, wq_ref[...],
                              preferred_element_type=jnp.float32)

    return pl.pallas_call(
        body,
        out_shape=(jax.ShapeDtypeStruct((_N, 128), jnp.float32),
                   jax.ShapeDtypeStruct((_N, 128), jnp.float32)),
    )(acc2, h2s, dinv, b2, g2, be2, whp, whq)


def _tc_final(r, s, bh1p, wh2p, bh2):
    blk = 8192

    def body(r_ref, s_ref, b_ref, w_ref, b2_ref, out_ref):
        h = jnp.maximum(r_ref[...] + s_ref[...] + b_ref[...], 0.0)
        acc = jnp.sum(h * w_ref[...], axis=1, keepdims=True) + b2_ref[0, 0]
        out_ref[...] = 1.0 / (1.0 + jnp.exp(-acc))

    grid = _P // blk
    return pl.pallas_call(
        body,
        grid=(grid,),
        in_specs=[
            pl.BlockSpec((blk, 128), lambda i: (i, 0)),
            pl.BlockSpec((blk, 128), lambda i: (i, 0)),
            pl.BlockSpec((1, 128), lambda i: (0, 0)),
            pl.BlockSpec((1, 128), lambda i: (0, 0)),
            pl.BlockSpec((1, 1), lambda i: (0, 0)),
        ],
        out_specs=pl.BlockSpec((blk, 1), lambda i: (i, 0)),
        out_shape=jax.ShapeDtypeStruct((_P, 1), jnp.float32),
    )(r, s, bh1p, wh2p, bh2)


def kernel(x, ei, src, dst, W1, b1, g1, be1, W2, b2, g2, be2,
           Wh1, bh1, Wh2, bh2):
    npd = _EPAD - _E
    pad_i = jnp.arange(npd, dtype=jnp.int32)
    src_pad = jnp.concatenate([ei[0], pad_i % _N])
    dst_pad = jnp.concatenate([ei[1], _N + pad_i % (_NPAD - _N)])
    src2d = src.reshape(-1, _CHUNK)
    dst2d = dst.reshape(-1, _CHUNK)

    ones128 = jnp.ones((_CHUNK, 128), jnp.float32)
    z128 = jnp.zeros((_ROWS_PT, 128), jnp.float32)

    w2p = jnp.concatenate([W2, jnp.zeros((128, 64), jnp.float32)], axis=1)
    b1r, g1r, be1r = (a.reshape(1, -1) for a in (b1, g1, be1))
    pad64 = jnp.zeros((1, 64), jnp.float32)
    b2p = jnp.concatenate([b2.reshape(1, -1), pad64], axis=1)
    g2p = jnp.concatenate([g2.reshape(1, -1), pad64], axis=1)
    be2p = jnp.concatenate([be2.reshape(1, -1), pad64], axis=1)
    zpad = jnp.zeros((64, 128), jnp.float32)
    whp = jnp.concatenate(
        [jnp.concatenate([Wh1[:64], Wh1[64:]], axis=1), zpad], axis=0)
    whq = jnp.concatenate(
        [jnp.concatenate([Wh1[64:], Wh1[:64]], axis=1), zpad], axis=0)
    bh1p = jnp.concatenate([bh1.reshape(1, -1), pad64], axis=1)
    wh2p = jnp.concatenate([Wh2.reshape(1, -1), pad64], axis=1)

    degp = _deg_kernel()(dst_pad, ones128, z128)
    dinv, h1s = _tc_prep(degp, x, W1)
    acc1 = _scatter_kernel()(h1s, src_pad, dst_pad, z128)
    h2s = _tc_mid(acc1, h1s, dinv, b1r, g1r, be1r, w2p)
    acc2 = _scatter_kernel()(h2s, src_pad, dst_pad, z128)
    pq, qp = _tc_head(acc2, h2s, dinv, b2p, g2p, be2p, whp, whq)
    r, s = _pairs_kernel()(pq, qp, src2d, dst2d)
    out = _tc_final(r, s, bh1p, wh2p, bh2.reshape(1, 1))
    return out[:, 0]


# speedup vs baseline: 23.6799x; 1.5326x over previous
import functools
import math

import jax
import jax.numpy as jnp
from jax import lax
from jax.experimental import pallas as pl
from jax.experimental.pallas import tpu as pltpu
from jax.experimental.pallas import tpu_sc as plsc

_N = 10000
_NPAD = 10112
_E = 320000
_NC = 2
_NS = 16
_NW = _NC * _NS
_CHUNK = 128
_EC = 64
_EPAD = 327680
_ECHUNKS = _EPAD // (_NW * _CHUNK)
_GCHUNKS = _EPAD // (_NW * _EC)
_NSLOTS = 4
_BLK = 32
_NB = _GCHUNKS // _BLK
_ROWS_PT = _NPAD // _NS
_P = 65536
_PCHUNKS = _P // (_NW * _CHUNK)
_BN_SCALE = 1.0 / math.sqrt(1.0 + 1e-5)

_mesh = plsc.VectorSubcoreMesh(core_axis_name="c", subcore_axis_name="s")


def _deg_kernel():
    @functools.partial(
        pl.kernel,
        out_type=jax.ShapeDtypeStruct((_NC, _NPAD, 128), jnp.float32),
        mesh=_mesh,
        scratch_types=[
            pltpu.VMEM((_ECHUNKS, _CHUNK), jnp.int32),
            pltpu.VMEM((_CHUNK, 128), jnp.float32),
            pltpu.VMEM_SHARED((_NPAD, 128), jnp.float32),
            pltpu.SemaphoreType.DMA,
            pltpu.SemaphoreType.DMA,
            pltpu.SemaphoreType.DMA,
            pltpu.SemaphoreType.DMA,
        ],
    )
    def k(dst_hbm, ones_hbm, z_hbm, out_hbm, dst_v, ones_v, acc_sh,
          s0, s1, s2, s3):
        c = lax.axis_index("c")
        s = lax.axis_index("s")
        wid = c * _NS + s
        row0 = s * _ROWS_PT
        sems = (s0, s1, s2, s3)
        pltpu.sync_copy(z_hbm, acc_sh.at[pl.ds(row0, _ROWS_PT)])
        pltpu.sync_copy(ones_hbm, ones_v)
        pltpu.sync_copy(dst_hbm.at[pl.ds(wid * _ECHUNKS, _ECHUNKS)], dst_v)
        plsc.subcore_barrier()

        for j in range(4):
            pltpu.async_copy(ones_v, acc_sh.at[dst_v.at[j]], sems[j],
                             add=True)

        def body(i, carry):
            for j in range(4):
                chunk = i + j
                pltpu.make_async_copy(ones_v, acc_sh.at[dst_v.at[chunk]],
                                      sems[j]).wait()

                @pl.when(chunk + 4 < _ECHUNKS)
                def _():
                    pltpu.async_copy(ones_v, acc_sh.at[dst_v.at[chunk + 4]],
                                     sems[j], add=True)
            return carry

        lax.fori_loop(0, _ECHUNKS // 4, lambda i, cy: body(i * 4, cy), 0)
        plsc.subcore_barrier()
        pltpu.sync_copy(acc_sh.at[pl.ds(row0, _ROWS_PT)],
                        out_hbm.at[c].at[pl.ds(row0, _ROWS_PT)])

    return k


def _scatter_kernel():
    ns = _NSLOTS
    ng = _BLK // ns
    scratch = (
        [pltpu.VMEM((_BLK, _EC), jnp.int32) for _ in range(4)]
        + [pltpu.VMEM((_EC, 128), jnp.float32) for _ in range(ns)]
        + [pltpu.SemaphoreType.DMA for _ in range(2 * ns + 4)]
        + [pltpu.VMEM_SHARED((_NPAD, 128), jnp.float32)]
    )

    @functools.partial(
        pl.kernel,
        out_type=jax.ShapeDtypeStruct((_NC, _NPAD, 128), jnp.float32),
        mesh=_mesh,
        scratch_types=scratch,
    )
    def k(h_hbm, src_hbm, dst_hbm, z_hbm, out_hbm, *scr):
        idxs = scr[0:2]
        idxd = scr[2:4]
        rows = scr[4:4 + ns]
        gsem = scr[4 + ns:4 + 2 * ns]
        ssem = scr[4 + 2 * ns:4 + 3 * ns]
        isem = scr[4 + 3 * ns:4 + 3 * ns + 2]
        dsem = scr[4 + 3 * ns + 2:4 + 3 * ns + 4]
        acc_sh = scr[-1]
        c = lax.axis_index("c")
        s = lax.axis_index("s")
        wid = c * _NS + s
        row0 = s * _ROWS_PT
        cbase = wid * _GCHUNKS
        pltpu.sync_copy(z_hbm, acc_sh.at[pl.ds(row0, _ROWS_PT)])

        def idx_copy(blk, p):
            a = pltpu.async_copy(
                src_hbm.at[pl.ds(cbase + blk * _BLK, _BLK)], idxs[p], isem[p])
            b = pltpu.async_copy(
                dst_hbm.at[pl.ds(cbase + blk * _BLK, _BLK)], idxd[p], dsem[p])
            return a, b

        def idx_wait(p):
            pltpu.make_async_copy(src_hbm.at[pl.ds(0, _BLK)], idxs[p],
                                  isem[p]).wait()
            pltpu.make_async_copy(dst_hbm.at[pl.ds(0, _BLK)], idxd[p],
                                  dsem[p]).wait()

        idx_copy(0, 0)
        idx_wait(0)
        idx_copy(1, 1)
        plsc.subcore_barrier()
        for j in range(ns):
            pltpu.async_copy(h_hbm.at[idxs[0].at[j]], rows[j], gsem[j])

        for bb in range(_NB):
            p = bb % 2
            for g in range(ng):
                if g == ng - 1 and bb + 1 < _NB:
                    idx_wait(1 - p)
                for j in range(ns):
                    r = g * ns + j
                    pltpu.make_async_copy(h_hbm.at[idxs[p].at[r]], rows[j],
                                          gsem[j]).wait()
                    pltpu.async_copy(rows[j], acc_sh.at[idxd[p].at[r]],
                                     ssem[j], add=True)
                    pltpu.make_async_copy(rows[j], acc_sh.at[idxd[p].at[r]],
                                          ssem[j]).wait()
                    if g < ng - 1:
                        pltpu.async_copy(h_hbm.at[idxs[p].at[r + ns]],
                                         rows[j], gsem[j])
                    elif bb + 1 < _NB:
                        pltpu.async_copy(h_hbm.at[idxs[1 - p].at[j]],
                                         rows[j], gsem[j])
            if bb + 2 < _NB:
                idx_copy(bb + 2, p)

        plsc.subcore_barrier()
        pltpu.sync_copy(acc_sh.at[pl.ds(row0, _ROWS_PT)],
                        out_hbm.at[c].at[pl.ds(row0, _ROWS_PT)])

    return k


def _pairs_kernel():
    sds = jax.ShapeDtypeStruct((_P, 128), jnp.float32)

    @functools.partial(
        pl.kernel,
        out_type=(sds, sds),
        mesh=_mesh,
        scratch_types=[
            pltpu.VMEM((_PCHUNKS, _CHUNK), jnp.int32),
            pltpu.VMEM((_PCHUNKS, _CHUNK), jnp.int32),
            pltpu.VMEM((_CHUNK, 128), jnp.float32),
            pltpu.VMEM((_CHUNK, 128), jnp.float32),
            pltpu.VMEM((_CHUNK, 128), jnp.float32),
            pltpu.VMEM((_CHUNK, 128), jnp.float32),
            pltpu.SemaphoreType.DMA,
            pltpu.SemaphoreType.DMA,
            pltpu.SemaphoreType.DMA,
            pltpu.SemaphoreType.DMA,
            pltpu.SemaphoreType.DMA,
            pltpu.SemaphoreType.DMA,
            pltpu.SemaphoreType.DMA,
            pltpu.SemaphoreType.DMA,
        ],
    )
    def k(pq_hbm, qp_hbm, src_hbm, dst_hbm, outr_hbm, outs_hbm,
          src_v, dst_v, bp0, bp1, bq0, bq1,
          gs0, gs1, gs2, gs3, ws0, ws1, ws2, ws3):
        bufp = (bp0, bp1)
        bufq = (bq0, bq1)
        gsem = (gs0, gs1, gs2, gs3)
        wsem = (ws0, ws1, ws2, ws3)
        c = lax.axis_index("c")
        s = lax.axis_index("s")
        wid = c * _NS + s
        base = wid * _PCHUNKS
        pltpu.sync_copy(src_hbm.at[pl.ds(base, _PCHUNKS)], src_v)
        pltpu.sync_copy(dst_hbm.at[pl.ds(base, _PCHUNKS)], dst_v)
        for j in range(2):
            pltpu.async_copy(pq_hbm.at[src_v.at[j]], bufp[j], gsem[j])
            pltpu.async_copy(qp_hbm.at[dst_v.at[j]], bufq[j], gsem[2 + j])

        def body(i, carry):
            for j in range(2):
                chunk = i + j
                off = pl.multiple_of((base + chunk) * _CHUNK, _CHUNK)
                pltpu.make_async_copy(pq_hbm.at[src_v.at[chunk]], bufp[j],
                                      gsem[j]).wait()
                pltpu.make_async_copy(qp_hbm.at[dst_v.at[chunk]], bufq[j],
                                      gsem[2 + j]).wait()
                pltpu.async_copy(bufp[j], outr_hbm.at[pl.ds(off, _CHUNK)],
                                 wsem[j])
                pltpu.async_copy(bufq[j], outs_hbm.at[pl.ds(off, _CHUNK)],
                                 wsem[2 + j])
                pltpu.make_async_copy(bufp[j], outr_hbm.at[pl.ds(off, _CHUNK)],
                                      wsem[j]).wait()
                pltpu.make_async_copy(bufq[j], outs_hbm.at[pl.ds(off, _CHUNK)],
                                      wsem[2 + j]).wait()

                @pl.when(chunk + 2 < _PCHUNKS)
                def _():
                    pltpu.async_copy(pq_hbm.at[src_v.at[chunk + 2]], bufp[j],
                                     gsem[j])
                    pltpu.async_copy(qp_hbm.at[dst_v.at[chunk + 2]], bufq[j],
                                     gsem[2 + j])
            return carry

        lax.fori_loop(0, _PCHUNKS // 2, lambda i, cy: body(i * 2, cy), 0)

    return k


def _tc_prep(degp, x, w1):
    def body(deg_ref, x_ref, w_ref, dinv_ref, h1s_ref):
        dinv = lax.rsqrt(deg_ref[0] + deg_ref[1] + 1.0)
        dinv_ref[...] = dinv
        h = jnp.dot(x_ref[...], w_ref[...],
                    preferred_element_type=jnp.float32)
        h1s_ref[...] = dinv[:_N] * h

    return pl.pallas_call(
        body,
        out_shape=(jax.ShapeDtypeStruct((_NPAD, 128), jnp.float32),
                   jax.ShapeDtypeStruct((_N, 128), jnp.float32)),
    )(degp, x, w1)


def _tc_mid(acc1, h1s, dinv, b1, g1, be1, w2p):
    def body(acc_ref, h_ref, dinv_ref, b_ref, g_ref, be_ref, w_ref, out_ref):
        dv = dinv_ref[:_N]
        o1 = dv * (acc_ref[0, :_N] + acc_ref[1, :_N] + h_ref[...]) + b_ref[...]
        z1 = jnp.maximum(g_ref[...] * (o1 * _BN_SCALE) + be_ref[...], 0.0)
        out_ref[...] = dv * jnp.dot(z1, w_ref[...],
                                    preferred_element_type=jnp.float32)

    return pl.pallas_call(
        body,
        out_shape=jax.ShapeDtypeStruct((_N, 128), jnp.float32),
    )(acc1, h1s, dinv, b1, g1, be1, w2p)


def _tc_head(acc2, h2s, dinv, b2, g2, be2, whp, whq):
    def body(acc_ref, h_ref, dinv_ref, b_ref, g_ref, be_ref, wp_ref, wq_ref,
             pq_ref, qp_ref):
        dv = dinv_ref[:_N]
        o2 = dv * (acc_ref[0, :_N] + acc_ref[1, :_N] + h_ref[...]) + b_ref[...]
        z2 = jnp.maximum(g_ref[...] * (o2 * _BN_SCALE) + be_ref[...], 0.0)
        pq_ref[...] = jnp.dot(z2, wp_ref[...],
                              preferred_element_type=jnp.float32)
        qp_ref[...] = jnp.dot(z2, wq_ref[...],
                              preferred_element_type=jnp.float32)

    return pl.pallas_call(
        body,
        out_shape=(jax.ShapeDtypeStruct((_N, 128), jnp.float32),
                   jax.ShapeDtypeStruct((_N, 128), jnp.float32)),
    )(acc2, h2s, dinv, b2, g2, be2, whp, whq)


def _tc_final(r, s, bh1p, wh2p, bh2):
    blk = 8192

    def body(r_ref, s_ref, b_ref, w_ref, b2_ref, out_ref):
        h = jnp.maximum(r_ref[...] + s_ref[...] + b_ref[...], 0.0)
        acc = jnp.sum(h * w_ref[...], axis=1, keepdims=True) + b2_ref[0, 0]
        out_ref[...] = 1.0 / (1.0 + jnp.exp(-acc))

    grid = _P // blk
    return pl.pallas_call(
        body,
        grid=(grid,),
        in_specs=[
            pl.BlockSpec((blk, 128), lambda i: (i, 0)),
            pl.BlockSpec((blk, 128), lambda i: (i, 0)),
            pl.BlockSpec((1, 128), lambda i: (0, 0)),
            pl.BlockSpec((1, 128), lambda i: (0, 0)),
            pl.BlockSpec((1, 1), lambda i: (0, 0)),
        ],
        out_specs=pl.BlockSpec((blk, 1), lambda i: (i, 0)),
        out_shape=jax.ShapeDtypeStruct((_P, 1), jnp.float32),
    )(r, s, bh1p, wh2p, bh2)


def kernel(x, ei, src, dst, W1, b1, g1, be1, W2, b2, g2, be2,
           Wh1, bh1, Wh2, bh2):
    npd = _EPAD - _E
    pad_i = jnp.arange(npd, dtype=jnp.int32)
    src_pad = jnp.concatenate([ei[0], pad_i % _N]).reshape(-1, _EC)
    dst_pad = jnp.concatenate([ei[1], _N + pad_i % (_NPAD - _N)])
    dst_deg = dst_pad.reshape(-1, _CHUNK)
    dst_pad = dst_pad.reshape(-1, _EC)
    src2d = src.reshape(-1, _CHUNK)
    dst2d = dst.reshape(-1, _CHUNK)

    ones128 = jnp.ones((_CHUNK, 128), jnp.float32)
    z128 = jnp.zeros((_ROWS_PT, 128), jnp.float32)

    w2p = jnp.concatenate([W2, jnp.zeros((128, 64), jnp.float32)], axis=1)
    b1r, g1r, be1r = (a.reshape(1, -1) for a in (b1, g1, be1))
    pad64 = jnp.zeros((1, 64), jnp.float32)
    b2p = jnp.concatenate([b2.reshape(1, -1), pad64], axis=1)
    g2p = jnp.concatenate([g2.reshape(1, -1), pad64], axis=1)
    be2p = jnp.concatenate([be2.reshape(1, -1), pad64], axis=1)
    zpad = jnp.zeros((64, 128), jnp.float32)
    whp = jnp.concatenate(
        [jnp.concatenate([Wh1[:64], Wh1[64:]], axis=1), zpad], axis=0)
    whq = jnp.concatenate(
        [jnp.concatenate([Wh1[64:], Wh1[:64]], axis=1), zpad], axis=0)
    bh1p = jnp.concatenate([bh1.reshape(1, -1), pad64], axis=1)
    wh2p = jnp.concatenate([Wh2.reshape(1, -1), pad64], axis=1)

    degp = _deg_kernel()(dst_deg, ones128, z128)
    dinv, h1s = _tc_prep(degp, x, W1)
    acc1 = _scatter_kernel()(h1s, src_pad, dst_pad, z128)
    h2s = _tc_mid(acc1, h1s, dinv, b1r, g1r, be1r, w2p)
    acc2 = _scatter_kernel()(h2s, src_pad, dst_pad, z128)
    pq, qp = _tc_head(acc2, h2s, dinv, b2p, g2p, be2p, whp, whq)
    r, s = _pairs_kernel()(pq, qp, src2d, dst2d)
    out = _tc_final(r, s, bh1p, wh2p, bh2.reshape(1, 1))
    return out[:, 0]


# speedup vs baseline: 26.0462x; 1.0999x over previous
import functools
import math

import numpy as _np

import jax
import jax.numpy as jnp
from jax import lax
from jax.experimental import pallas as pl
from jax.experimental.pallas import tpu as pltpu
from jax.experimental.pallas import tpu_sc as plsc

_N = 10000
_NPAD = 10112
_E = 320000
_NC = 2
_NS = 16
_NW = _NC * _NS
_CHUNK = 128
_EC = 64
_EPAD = 327680
_ECHUNKS = _EPAD // (_NW * _CHUNK)
_GCHUNKS = _EPAD // (_NW * _EC)
_NSLOTS = 4
_BLK = 32
_NB = _GCHUNKS // _BLK
_ROWS_PT = _NPAD // _NS
_P = 65536
_PCHUNKS = _P // (_NW * _CHUNK)
_BN_SCALE = 1.0 / math.sqrt(1.0 + 1e-5)

_mesh = plsc.VectorSubcoreMesh(core_axis_name="c", subcore_axis_name="s")

_pad_i = _np.arange(_EPAD - _E, dtype=_np.int32)
_PAD_SRC = (_pad_i % _N).reshape(-1, _EC)
_PAD_DST = (_N + _pad_i % (_NPAD - _N)).reshape(-1, _EC)


def _deg_kernel():
    @functools.partial(
        pl.kernel,
        out_type=jax.ShapeDtypeStruct((_NC, _NPAD, 16), jnp.float32),
        mesh=_mesh,
        scratch_types=[
            pltpu.VMEM((_GCHUNKS, _EC), jnp.int32),
            pltpu.VMEM((_EC, 16), jnp.float32),
            pltpu.VMEM_SHARED((_NPAD, 16), jnp.float32),
            pltpu.SemaphoreType.DMA,
            pltpu.SemaphoreType.DMA,
            pltpu.SemaphoreType.DMA,
            pltpu.SemaphoreType.DMA,
        ],
    )
    def k(dst_hbm, ones_hbm, z_hbm, out_hbm, dst_v, ones_v, acc_sh,
          s0, s1, s2, s3):
        c = lax.axis_index("c")
        s = lax.axis_index("s")
        wid = c * _NS + s
        row0 = s * _ROWS_PT
        sems = (s0, s1, s2, s3)
        pltpu.sync_copy(z_hbm, acc_sh.at[pl.ds(row0, _ROWS_PT)])
        pltpu.sync_copy(ones_hbm, ones_v)
        pltpu.sync_copy(dst_hbm.at[pl.ds(wid * _GCHUNKS, _GCHUNKS)], dst_v)
        plsc.subcore_barrier()

        for j in range(4):
            pltpu.async_copy(ones_v, acc_sh.at[dst_v.at[j]], sems[j],
                             add=True)

        def body(i, carry):
            for j in range(4):
                chunk = i + j
                pltpu.make_async_copy(ones_v, acc_sh.at[dst_v.at[chunk]],
                                      sems[j]).wait()

                @pl.when(chunk + 4 < _GCHUNKS)
                def _():
                    pltpu.async_copy(ones_v, acc_sh.at[dst_v.at[chunk + 4]],
                                     sems[j], add=True)
            return carry

        lax.fori_loop(0, _GCHUNKS // 4, lambda i, cy: body(i * 4, cy), 0)
        plsc.subcore_barrier()
        pltpu.sync_copy(acc_sh.at[pl.ds(row0, _ROWS_PT)],
                        out_hbm.at[c].at[pl.ds(row0, _ROWS_PT)])

    return k


def _scatter_kernel():
    ns = _NSLOTS
    ng = _BLK // ns
    scratch = (
        [pltpu.VMEM((_BLK, _EC), jnp.int32) for _ in range(4)]
        + [pltpu.VMEM((_EC, 128), jnp.float32) for _ in range(ns)]
        + [pltpu.SemaphoreType.DMA for _ in range(2 * ns + 4)]
        + [pltpu.VMEM_SHARED((_NPAD, 128), jnp.float32)]
    )

    @functools.partial(
        pl.kernel,
        out_type=jax.ShapeDtypeStruct((_NC, _NPAD, 128), jnp.float32),
        mesh=_mesh,
        scratch_types=scratch,
    )
    def k(h_hbm, src_hbm, dst_hbm, z_hbm, out_hbm, *scr):
        idxs = scr[0:2]
        idxd = scr[2:4]
        rows = scr[4:4 + ns]
        gsem = scr[4 + ns:4 + 2 * ns]
        ssem = scr[4 + 2 * ns:4 + 3 * ns]
        isem = scr[4 + 3 * ns:4 + 3 * ns + 2]
        dsem = scr[4 + 3 * ns + 2:4 + 3 * ns + 4]
        acc_sh = scr[-1]
        c = lax.axis_index("c")
        s = lax.axis_index("s")
        wid = c * _NS + s
        row0 = s * _ROWS_PT
        cbase = wid * _GCHUNKS
        pltpu.sync_copy(z_hbm, acc_sh.at[pl.ds(row0, _ROWS_PT)])

        def idx_copy(blk, p):
            a = pltpu.async_copy(
                src_hbm.at[pl.ds(cbase + blk * _BLK, _BLK)], idxs[p], isem[p])
            b = pltpu.async_copy(
                dst_hbm.at[pl.ds(cbase + blk * _BLK, _BLK)], idxd[p], dsem[p])
            return a, b

        def idx_wait(p):
            pltpu.make_async_copy(src_hbm.at[pl.ds(0, _BLK)], idxs[p],
                                  isem[p]).wait()
            pltpu.make_async_copy(dst_hbm.at[pl.ds(0, _BLK)], idxd[p],
                                  dsem[p]).wait()

        idx_copy(0, 0)
        idx_wait(0)
        idx_copy(1, 1)
        plsc.subcore_barrier()
        for j in range(ns):
            pltpu.async_copy(h_hbm.at[idxs[0].at[j]], rows[j], gsem[j])

        for bb in range(_NB):
            p = bb % 2
            for g in range(ng):
                if g == ng - 1 and bb + 1 < _NB:
                    idx_wait(1 - p)
                for j in range(ns):
                    r = g * ns + j
                    pltpu.make_async_copy(h_hbm.at[idxs[p].at[r]], rows[j],
                                          gsem[j]).wait()
                    pltpu.async_copy(rows[j], acc_sh.at[idxd[p].at[r]],
                                     ssem[j], add=True)
                    pltpu.make_async_copy(rows[j], acc_sh.at[idxd[p].at[r]],
                                          ssem[j]).wait()
                    if g < ng - 1:
                        pltpu.async_copy(h_hbm.at[idxs[p].at[r + ns]],
                                         rows[j], gsem[j])
                    elif bb + 1 < _NB:
                        pltpu.async_copy(h_hbm.at[idxs[1 - p].at[j]],
                                         rows[j], gsem[j])
            if bb + 2 < _NB:
                idx_copy(bb + 2, p)

        plsc.subcore_barrier()
        pltpu.sync_copy(acc_sh.at[pl.ds(row0, _ROWS_PT)],
                        out_hbm.at[c].at[pl.ds(row0, _ROWS_PT)])

    return k


def _pairs_kernel():
    sds = jax.ShapeDtypeStruct((_P, 128), jnp.float32)

    npc = _P // (_NW * _EC)
    scratch = (
        [pltpu.VMEM((npc, _EC), jnp.int32) for _ in range(2)]
        + [pltpu.VMEM((_EC, 128), jnp.float32) for _ in range(8)]
        + [pltpu.SemaphoreType.DMA for _ in range(16)]
    )

    @functools.partial(
        pl.kernel,
        out_type=(sds, sds),
        mesh=_mesh,
        scratch_types=scratch,
    )
    def k(pq_hbm, qp_hbm, src_hbm, dst_hbm, outr_hbm, outs_hbm, *scr):
        src_v, dst_v = scr[0], scr[1]
        bufp = scr[2:6]
        bufq = scr[6:10]
        gpsem = scr[10:14]
        gqsem = scr[14:18]
        wrsem = scr[18:22]
        wssem = scr[22:26]
        c = lax.axis_index("c")
        s = lax.axis_index("s")
        wid = c * _NS + s
        base = wid * npc
        pltpu.sync_copy(src_hbm.at[pl.ds(base, npc)], src_v)
        pltpu.sync_copy(dst_hbm.at[pl.ds(base, npc)], dst_v)
        for j in range(4):
            pltpu.async_copy(pq_hbm.at[src_v.at[j]], bufp[j], gpsem[j])
            pltpu.async_copy(qp_hbm.at[dst_v.at[j]], bufq[j], gqsem[j])

        def body(i, carry):
            for j in range(4):
                chunk = i + j
                off = pl.multiple_of((base + chunk) * _EC, _EC)
                pltpu.make_async_copy(pq_hbm.at[src_v.at[chunk]], bufp[j],
                                      gpsem[j]).wait()
                pltpu.make_async_copy(qp_hbm.at[dst_v.at[chunk]], bufq[j],
                                      gqsem[j]).wait()
                pltpu.async_copy(bufp[j], outr_hbm.at[pl.ds(off, _EC)],
                                 wrsem[j])
                pltpu.async_copy(bufq[j], outs_hbm.at[pl.ds(off, _EC)],
                                 wssem[j])
                pltpu.make_async_copy(bufp[j], outr_hbm.at[pl.ds(off, _EC)],
                                      wrsem[j]).wait()
                pltpu.make_async_copy(bufq[j], outs_hbm.at[pl.ds(off, _EC)],
                                      wssem[j]).wait()

                @pl.when(chunk + 4 < npc)
                def _():
                    pltpu.async_copy(pq_hbm.at[src_v.at[chunk + 4]], bufp[j],
                                     gpsem[j])
                    pltpu.async_copy(qp_hbm.at[dst_v.at[chunk + 4]], bufq[j],
                                     gqsem[j])
            return carry

        lax.fori_loop(0, npc // 4, lambda i, cy: body(i * 4, cy), 0)

    return k


def _tc_prep(degp, x, w1):
    def body(deg_ref, x_ref, w_ref, dinv_ref, h1s_ref):
        dinv16 = lax.rsqrt(deg_ref[0] + deg_ref[1] + 1.0)
        dinv = dinv16[:, 0:1]
        dinv_ref[...] = dinv
        h = jnp.dot(x_ref[...], w_ref[...],
                    preferred_element_type=jnp.float32)
        h1s_ref[...] = dinv[:_N] * h

    return pl.pallas_call(
        body,
        out_shape=(jax.ShapeDtypeStruct((_NPAD, 1), jnp.float32),
                   jax.ShapeDtypeStruct((_N, 128), jnp.float32)),
    )(degp, x, w1)


def _tc_mid(acc1, h1s, dinv, b1, g1, be1, w2p):
    def body(acc_ref, h_ref, dinv_ref, b_ref, g_ref, be_ref, w_ref, out_ref):
        dv = dinv_ref[:_N]
        o1 = dv * (acc_ref[0, :_N] + acc_ref[1, :_N] + h_ref[...]) + b_ref[...]
        z1 = jnp.maximum(g_ref[...] * (o1 * _BN_SCALE) + be_ref[...], 0.0)
        out_ref[...] = dv * jnp.dot(z1, w_ref[...],
                                    preferred_element_type=jnp.float32)

    return pl.pallas_call(
        body,
        out_shape=jax.ShapeDtypeStruct((_N, 128), jnp.float32),
    )(acc1, h1s, dinv, b1, g1, be1, w2p)


def _tc_head(acc2, h2s, dinv, b2, g2, be2, whp, whq):
    def body(acc_ref, h_ref, dinv_ref, b_ref, g_ref, be_ref, wp_ref, wq_ref,
             pq_ref, qp_ref):
        dv = dinv_ref[:_N]
        o2 = dv * (acc_ref[0, :_N] + acc_ref[1, :_N] + h_ref[...]) + b_ref[...]
        z2 = jnp.maximum(g_ref[...] * (o2 * _BN_SCALE) + be_ref[...], 0.0)
        pq_ref[...] = jnp.dot(z2, wp_ref[...],
                              preferred_element_type=jnp.float32)
        qp_ref[...] = jnp.dot(z2, wq_ref[...],
                              preferred_element_type=jnp.float32)

    return pl.pallas_call(
        body,
        out_shape=(jax.ShapeDtypeStruct((_N, 128), jnp.float32),
                   jax.ShapeDtypeStruct((_N, 128), jnp.float32)),
    )(acc2, h2s, dinv, b2, g2, be2, whp, whq)


def _tc_final(r, s, bh1p, wh2p, bh2):
    blk = 8192

    def body(r_ref, s_ref, b_ref, w_ref, b2_ref, out_ref):
        h = jnp.maximum(r_ref[...] + s_ref[...] + b_ref[...], 0.0)
        acc = jnp.sum(h * w_ref[...], axis=1, keepdims=True) + b2_ref[0, 0]
        out_ref[...] = 1.0 / (1.0 + jnp.exp(-acc))

    grid = _P // blk
    return pl.pallas_call(
        body,
        grid=(grid,),
        in_specs=[
            pl.BlockSpec((blk, 128), lambda i: (i, 0)),
            pl.BlockSpec((blk, 128), lambda i: (i, 0)),
            pl.BlockSpec((1, 128), lambda i: (0, 0)),
            pl.BlockSpec((1, 128), lambda i: (0, 0)),
            pl.BlockSpec((1, 1), lambda i: (0, 0)),
        ],
        out_specs=pl.BlockSpec((blk, 1), lambda i: (i, 0)),
        out_shape=jax.ShapeDtypeStruct((_P, 1), jnp.float32),
    )(r, s, bh1p, wh2p, bh2)


def kernel(x, ei, src, dst, W1, b1, g1, be1, W2, b2, g2, be2,
           Wh1, bh1, Wh2, bh2):
    src_pad = jnp.concatenate([ei[0].reshape(-1, _EC), _PAD_SRC])
    dst_pad = jnp.concatenate([ei[1].reshape(-1, _EC), _PAD_DST])
    src2d = src.reshape(-1, _EC)
    dst2d = dst.reshape(-1, _EC)

    ones16 = jnp.ones((_EC, 16), jnp.float32)
    z16 = jnp.zeros((_ROWS_PT, 16), jnp.float32)
    z128 = jnp.zeros((_ROWS_PT, 128), jnp.float32)

    w2p = jnp.concatenate([W2, jnp.zeros((128, 64), jnp.float32)], axis=1)
    b1r, g1r, be1r = (a.reshape(1, -1) for a in (b1, g1, be1))
    pad64 = jnp.zeros((1, 64), jnp.float32)
    b2p = jnp.concatenate([b2.reshape(1, -1), pad64], axis=1)
    g2p = jnp.concatenate([g2.reshape(1, -1), pad64], axis=1)
    be2p = jnp.concatenate([be2.reshape(1, -1), pad64], axis=1)
    zpad = jnp.zeros((64, 128), jnp.float32)
    whp = jnp.concatenate(
        [jnp.concatenate([Wh1[:64], Wh1[64:]], axis=1), zpad], axis=0)
    whq = jnp.concatenate(
        [jnp.concatenate([Wh1[64:], Wh1[:64]], axis=1), zpad], axis=0)
    bh1p = jnp.concatenate([bh1.reshape(1, -1), pad64], axis=1)
    wh2p = jnp.concatenate([Wh2.reshape(1, -1), pad64], axis=1)

    degp = _deg_kernel()(dst_pad, ones16, z16)
    dinv, h1s = _tc_prep(degp, x, W1)
    acc1 = _scatter_kernel()(h1s, src_pad, dst_pad, z128)
    h2s = _tc_mid(acc1, h1s, dinv, b1r, g1r, be1r, w2p)
    acc2 = _scatter_kernel()(h2s, src_pad, dst_pad, z128)
    pq, qp = _tc_head(acc2, h2s, dinv, b2p, g2p, be2p, whp, whq)
    r, s = _pairs_kernel()(pq, qp, src2d, dst2d)
    out = _tc_final(r, s, bh1p, wh2p, bh2.reshape(1, 1))
    return out.reshape(_P)
